# Initial kernel scaffold; baseline (speedup 1.0000x reference)
import jax
import jax.numpy as jnp
from jax.experimental import pallas as pl


def kernel(x, hyperedge_index, weight):
    raise NotImplementedError("write your pallas kernel here")


# speedup vs baseline: 2.3554x; 2.3554x over previous
import jax
import jax.numpy as jnp
from jax import lax
from jax.experimental import pallas as pl
from jax.experimental.pallas import tpu as pltpu
from jax.experimental.pallas import tpu_sc as plsc


def _mm_body(x_ref, w_ref, o_ref):
  o_ref[0] = jnp.dot(x_ref[0], w_ref[...],
                     preferred_element_type=jnp.float32)


def _matmul(x, w):
  B, S, FIN = x.shape
  F = w.shape[1]
  SB = 512
  return pl.pallas_call(
      _mm_body,
      grid=(B, S // SB),
      in_specs=[
          pl.BlockSpec((1, SB, FIN), lambda b, i: (b, i, 0)),
          pl.BlockSpec((FIN, F), lambda b, i: (0, 0)),
      ],
      out_specs=pl.BlockSpec((1, SB, F), lambda b, i: (b, i, 0)),
      out_shape=jax.ShapeDtypeStruct((B, S, F), jnp.float32),
  )(x, w)


def _loss_body(n_ref, a_ref, x_ref, o_ref):
  i = pl.program_id(0)
  s = jnp.sum(jnp.abs(a_ref[...] - x_ref[...]))
  prev = jnp.where(i == 0, 0.0, o_ref[0, 0])
  tot = prev + s
  o_ref[0, 0] = jnp.where(i == pl.num_programs(0) - 1,
                          tot / n_ref[0], tot)


def _loss(agg_flat, xt_flat):
  N, F = agg_flat.shape
  RB = 1024
  n = jnp.full((1,), float(N * F), dtype=jnp.float32)
  out = pl.pallas_call(
      _loss_body,
      grid=(N // RB,),
      in_specs=[
          pl.BlockSpec(memory_space=pltpu.SMEM),
          pl.BlockSpec((RB, F), lambda i: (i, 0)),
          pl.BlockSpec((RB, F), lambda i: (i, 0)),
      ],
      out_specs=pl.BlockSpec(memory_space=pltpu.SMEM),
      out_shape=jax.ShapeDtypeStruct((1, 1), jnp.float32),
  )(n, agg_flat, xt_flat)
  return out[0, 0]


_B, _S, _F, _E = 4, 8192, 256, 32768
_NT = 16
_EPT = _E // _NT
_NPT = _S // _NT
_WIN = 128
_NWIN = _S // _WIN
_NPASS = _NWIN // _NT
_TRASH_E = _WIN
_BS = _B * _S
_PCH = 32
_PROWS = _EPT // _PCH + _NWIN
_PLSZ = _PROWS * _PCH
_OROWS = _NPT // 16 + _NWIN
_OSZ = _OROWS * 16
_LANES = 16


def _sc_body(xt_hbm, nidx_hbm, eidx_hbm, agg_hbm,
             nbuf, ebuf2, ebuff, ones_v, zvec,
             cnt_v, best_v, mrg_v,
             nlall, elall, onall, oeall, ovall, onone,
             gnb, geb, gidx, onb, oeb, ovb,
             rows, acc, ob, smem,
             hist_sp, stage_sp, pn_sp, pe_sp, on_sp, oe_sp, ov_sp,
             sem_g, sem_o):
  c = lax.axis_index("c")
  s = lax.axis_index("s")

  pltpu.sync_copy(nidx_hbm.at[pl.ds(s * _EPT, _EPT)], nbuf)
  pltpu.sync_copy(eidx_hbm.at[pl.ds(s * _EPT, _EPT)], ebuff)
  for j in range(16):
    pltpu.sync_copy(eidx_hbm.at[pl.ds(s * _EPT + j * 128, 128)], ebuf2.at[j])

  zf = jnp.zeros((_LANES,), jnp.float32)

  def _init1(i, _):
    ones_v[pl.ds(i * 16, 16)] = jnp.ones((16,), jnp.float32)
    zvec[pl.ds(i * 16, 16)] = zf
    return 0
  lax.fori_loop(0, 32, _init1, 0)

  def _init2(i, _):
    best_v[pl.ds(i * 16, 16)] = jnp.full((16,), -1, jnp.int32)
    return 0
  lax.fori_loop(0, 512, _init2, 0)

  padn = jnp.zeros((16,), jnp.int32)
  pade = jnp.full((16,), _TRASH_E, jnp.int32)
  padbig = jnp.full((16,), 1 << 20, jnp.int32)

  def _init4(i, _):
    nlall[pl.ds(i * 16, 16)] = padn
    elall[pl.ds(i * 16, 16)] = pade
    return 0
  lax.fori_loop(0, _PLSZ // 16, _init4, 0)

  def _init5(i, _):
    onall[pl.ds(i * 16, 16)] = padbig
    oeall[pl.ds(i * 16, 16)] = pade
    ovall[pl.ds(i * 16, 16)] = zf
    return 0
  lax.fori_loop(0, _OSZ // 16, _init5, 0)

  def _init6(i, _):
    onone[pl.ds(i * 16, 16)] = padbig
    return 0
  lax.fori_loop(0, (_NPT + 16) // 16, _init6, 0)

  pltpu.sync_copy(zvec, hist_sp.at[pl.ds(s * 512, 512)])
  plsc.subcore_barrier()

  def _hist(j, _):
    pltpu.sync_copy(ones_v.at[pl.ds(0, 128)], hist_sp.at[ebuf2.at[j]],
                    add=True)
    return 0
  lax.fori_loop(0, 16, _hist, 0)
  plsc.subcore_barrier()

  pltpu.sync_copy(hist_sp, cnt_v)

  iot = lax.iota(jnp.int32, _LANES)

  def _pairs_a(g, cnts):
    n16 = nbuf[pl.ds(g * 16, 16)]
    e16 = ebuff[pl.ds(g * 16, 16)]
    cvals = plsc.load_gather(cnt_v, [e16])
    elig = cvals > 1.5
    cand = jnp.where(elig, e16, -1)
    comb = n16 * 16384 + (cand + 1)
    sk, _sv = plsc.sort_key_val(comb, comb)
    ns = lax.shift_right_logical(sk, 14)
    cs = (sk & 16383) - 1
    _, lastm = plsc.scan_count(ns)
    cur = plsc.load_gather(best_v, [ns])
    plsc.store_scatter(best_v, [ns], jnp.maximum(cur, cs), mask=lastm)
    w16 = lax.shift_right_logical(e16, 7)
    out = []
    for k in range(_NWIN):
      pck = plsc.all_reduce_population_count(w16 == k)[0]
      out.append(cnts[k] + pck)
    return tuple(out)

  cnt_p = lax.fori_loop(0, _EPT // 16, _pairs_a, (0,) * _NWIN)

  prow = []
  r0 = 0
  for k in range(_NWIN):
    prow.append(r0)
    r0 = r0 + (cnt_p[k] + _PCH - 1) // _PCH

  def _pairs_b(g, offs):
    n16 = nbuf[pl.ds(g * 16, 16)]
    e16 = ebuff[pl.ds(g * 16, 16)]
    w16 = lax.shift_right_logical(e16, 7)
    eloc16 = e16 & (_WIN - 1)
    out = []
    for k in range(_NWIN):
      mk = w16 == k
      pck = plsc.all_reduce_population_count(mk)[0]
      plsc.store_compressed(nlall.at[pl.ds(offs[k], 16)], n16, mask=mk)
      plsc.store_compressed(elall.at[pl.ds(offs[k], 16)], eloc16, mask=mk)
      out.append(offs[k] + pck)
    return tuple(out)

  lax.fori_loop(0, _EPT // 16, _pairs_b, tuple(r * _PCH for r in prow))

  pltpu.sync_copy(best_v, stage_sp.at[s])
  plsc.subcore_barrier()
  pltpu.sync_copy(stage_sp.at[:, pl.ds(s * _NPT, _NPT)], mrg_v)

  def _nodes_a(g, cnts):
    m = mrg_v[0, pl.ds(g * 16, 16)]
    for r in range(1, 16):
      m = jnp.maximum(m, mrg_v[r, pl.ds(g * 16, 16)])
    best_v[pl.ds(g * 16, 16)] = m
    mw = lax.shift_right_logical(jnp.maximum(m, 0), 7)
    valid = m >= 0
    out = []
    for k in range(_NWIN):
      pck = plsc.all_reduce_population_count(valid & (mw == k))[0]
      out.append(cnts[k] + pck)
    pcn = plsc.all_reduce_population_count(m < 0)[0]
    return tuple(out) + (cnts[_NWIN] + pcn,)

  cnt_o = lax.fori_loop(0, _NPT // 16, _nodes_a, (0,) * (_NWIN + 1))

  orow = []
  r0 = 0
  for k in range(_NWIN):
    orow.append(r0)
    r0 = r0 + (cnt_o[k] + 15) // 16

  def _nodes_b(g, offs):
    m = best_v[pl.ds(g * 16, 16)]
    nodeid = s * _NPT + g * 16 + iot
    cb = plsc.load_gather(cnt_v, [jnp.maximum(m, 0)])
    iv = 1.0 / jnp.maximum(cb, 1.0)
    valid = m >= 0
    mw = lax.shift_right_logical(jnp.maximum(m, 0), 7)
    mloc = jnp.maximum(m, 0) & (_WIN - 1)
    out = []
    for k in range(_NWIN):
      mk = valid & (mw == k)
      pck = plsc.all_reduce_population_count(mk)[0]
      plsc.store_compressed(onall.at[pl.ds(offs[k], 16)], nodeid, mask=mk)
      plsc.store_compressed(oeall.at[pl.ds(offs[k], 16)], mloc, mask=mk)
      plsc.store_compressed(ovall.at[pl.ds(offs[k], 16)], iv, mask=mk)
      out.append(offs[k] + pck)
    mn = m < 0
    pcn = plsc.all_reduce_population_count(mn)[0]
    plsc.store_compressed(onone.at[pl.ds(offs[_NWIN], 16)], nodeid, mask=mn)
    return tuple(out) + (offs[_NWIN] + pcn,)

  lax.fori_loop(0, _NPT // 16, _nodes_b,
                tuple(r * 16 for r in orow) + (0,))
  cnt_on = cnt_o[_NWIN]

  for k in range(_NWIN):
    smem[k] = prow[k]
    smem[_NWIN + k] = cnt_p[k]
    smem[2 * _NWIN + k] = orow[k]
    smem[3 * _NWIN + k] = cnt_o[k]

  pltpu.sync_copy(nlall, pn_sp.at[s])
  pltpu.sync_copy(elall, pe_sp.at[s])
  pltpu.sync_copy(onall, on_sp.at[s])
  pltpu.sync_copy(oeall, oe_sp.at[s])
  pltpu.sync_copy(ovall, ov_sp.at[s])
  plsc.subcore_barrier()

  def _pf(p, _):
    w = p * _NT + s

    def _pfs(st, _s):
      base = 256 + p * 64 + st * 4
      smem[base + 0] = plsc.fetch_and_add(smem.at[w], 0, subcore_id=st)
      smem[base + 1] = plsc.fetch_and_add(smem.at[_NWIN + w], 0,
                                          subcore_id=st)
      smem[base + 2] = plsc.fetch_and_add(smem.at[2 * _NWIN + w], 0,
                                          subcore_id=st)
      smem[base + 3] = plsc.fetch_and_add(smem.at[3 * _NWIN + w], 0,
                                          subcore_id=st)
      return 0
    lax.fori_loop(0, _NT, _pfs, 0)
    return 0
  lax.fori_loop(0, _NPASS, _pf, 0)
  plsc.subcore_barrier()

  def _none_pass(bi, _):
    bbase = (c * 2 + bi) * _S

    def _none(v, _v):
      nn = onone[pl.ds(v * 16, 16)]
      src = jnp.where(nn >= _S, 0, nn + bbase)
      dst = jnp.where(nn >= _S, _BS, nn + bbase)
      pltpu.async_copy(xt_hbm.at[src], ob, sem_o).wait()
      pltpu.async_copy(ob, agg_hbm.at[dst], sem_o).wait()
      return 0
    lax.fori_loop(0, (cnt_on + 15) // 16, _none, 0)
    return 0
  lax.fori_loop(0, 2, _none_pass, 0)

  def _bp(t, _):
    bi = lax.div(t, _NPASS)
    p = t - bi * _NPASS
    bbase = (c * 2 + bi) * _S

    def _zero(i, _i):
      for k in range(16):
        acc[i, pl.ds(k * 16, 16)] = zf
      return 0
    lax.fori_loop(0, _WIN + 1, _zero, 0)

    def _src(st, _st):
      base = 256 + p * 64 + st * 4
      rs = smem[base + 0]
      cp = smem[base + 1]

      def _chunk(q, _q):
        off = (rs + q) * _PCH
        pltpu.sync_copy(pn_sp.at[st, pl.ds(off, _PCH)], gnb)
        pltpu.sync_copy(pe_sp.at[st, pl.ds(off, _PCH)], geb)
        for m in range(_PCH // 16):
          gidx[pl.ds(m * 16, 16)] = gnb[pl.ds(m * 16, 16)] + bbase
        pltpu.async_copy(xt_hbm.at[gidx], rows, sem_g).wait()

        def _addrow(rr, _r):
          el16 = geb[pl.ds(rr * 16, 16)]
          for r in range(16):
            e = el16[r]
            for cg in range(16):
              plsc.addupdate(acc.at[e, pl.ds(cg * 16, 16)],
                             rows[rr * 16 + r, pl.ds(cg * 16, 16)])
          return 0
        lax.fori_loop(0, _PCH // 16, _addrow, 0)
        return 0
      lax.fori_loop(0, (cp + _PCH - 1) // _PCH, _chunk, 0)
      return 0
    lax.fori_loop(0, _NT, _src, 0)

    def _osrc(st, _st):
      base = 256 + p * 64 + st * 4
      ros = smem[base + 2]
      co = smem[base + 3]

      def _ochunk(v, _v):
        off = (ros + v) * 16
        pltpu.sync_copy(on_sp.at[st, pl.ds(off, 16)], onb)
        pltpu.sync_copy(oe_sp.at[st, pl.ds(off, 16)], oeb)
        pltpu.sync_copy(ov_sp.at[st, pl.ds(off, 16)], ovb)
        nn = onb[...]
        el16 = oeb[...]
        iv16 = ovb[...]
        dst = jnp.where(nn >= _S, _BS, nn + bbase)
        for r in range(16):
          e = el16[r]
          sc = iv16[r]
          for cg in range(16):
            ob[r, pl.ds(cg * 16, 16)] = acc[e, pl.ds(cg * 16, 16)] * sc
        pltpu.async_copy(ob, agg_hbm.at[dst], sem_o).wait()
        return 0
      lax.fori_loop(0, (co + 15) // 16, _ochunk, 0)
      return 0
    lax.fori_loop(0, _NT, _osrc, 0)
    return 0

  lax.fori_loop(0, 2 * _NPASS, _bp, 0)


def _sc_gather_scatter(xt_flat, node_idx, edge_idx):
  mesh = plsc.VectorSubcoreMesh(core_axis_name="c", subcore_axis_name="s")
  f = pl.kernel(
      _sc_body,
      out_type=jax.ShapeDtypeStruct((_BS + 64, _F), jnp.float32),
      mesh=mesh,
      compiler_params=pltpu.CompilerParams(needs_layout_passes=False),
      scratch_types=[
          pltpu.VMEM((_EPT,), jnp.int32),
          pltpu.VMEM((16, 128), jnp.int32),
          pltpu.VMEM((_EPT,), jnp.int32),
          pltpu.VMEM((512,), jnp.float32),
          pltpu.VMEM((512,), jnp.float32),
          pltpu.VMEM((_S,), jnp.float32),
          pltpu.VMEM((_S,), jnp.int32),
          pltpu.VMEM((16, _NPT), jnp.int32),
          pltpu.VMEM((_PLSZ,), jnp.int32),
          pltpu.VMEM((_PLSZ,), jnp.int32),
          pltpu.VMEM((_OSZ,), jnp.int32),
          pltpu.VMEM((_OSZ,), jnp.int32),
          pltpu.VMEM((_OSZ,), jnp.float32),
          pltpu.VMEM((_NPT + 16,), jnp.int32),
          pltpu.VMEM((_PCH,), jnp.int32),
          pltpu.VMEM((_PCH,), jnp.int32),
          pltpu.VMEM((_PCH,), jnp.int32),
          pltpu.VMEM((16,), jnp.int32),
          pltpu.VMEM((16,), jnp.int32),
          pltpu.VMEM((16,), jnp.float32),
          pltpu.VMEM((_PCH, _F), jnp.float32),
          pltpu.VMEM((_WIN + 1, _F), jnp.float32),
          pltpu.VMEM((16, _F), jnp.float32),
          pltpu.SMEM((544,), jnp.int32),
          pltpu.VMEM_SHARED((_S,), jnp.float32),
          pltpu.VMEM_SHARED((16, _S), jnp.int32),
          pltpu.VMEM_SHARED((16, _PLSZ), jnp.int32),
          pltpu.VMEM_SHARED((16, _PLSZ), jnp.int32),
          pltpu.VMEM_SHARED((16, _OSZ), jnp.int32),
          pltpu.VMEM_SHARED((16, _OSZ), jnp.int32),
          pltpu.VMEM_SHARED((16, _OSZ), jnp.float32),
          pltpu.SemaphoreType.DMA,
          pltpu.SemaphoreType.DMA,
      ],
  )
  return f(xt_flat, node_idx, edge_idx)


def kernel(x, hyperedge_index, weight):
  xt = _matmul(x, weight)
  xt_flat = xt.reshape(_BS, _F)
  node_idx = hyperedge_index[0]
  edge_idx = hyperedge_index[1]
  agg_pad = _sc_gather_scatter(xt_flat, node_idx, edge_idx)
  agg = agg_pad[:_BS].reshape(_B, _S, _F)
  loss = _loss(agg_pad[:_BS], xt_flat)
  return agg, loss


# speedup vs baseline: 2.4465x; 1.0387x over previous
import jax
import jax.numpy as jnp
from jax import lax
from jax.experimental import pallas as pl
from jax.experimental.pallas import tpu as pltpu
from jax.experimental.pallas import tpu_sc as plsc


def _mm_body(x_ref, w_ref, o_ref):
  o_ref[0] = jnp.dot(x_ref[0], w_ref[...],
                     preferred_element_type=jnp.float32)


def _matmul(x, w):
  B, S, FIN = x.shape
  F = w.shape[1]
  SB = 512
  return pl.pallas_call(
      _mm_body,
      grid=(B, S // SB),
      in_specs=[
          pl.BlockSpec((1, SB, FIN), lambda b, i: (b, i, 0)),
          pl.BlockSpec((FIN, F), lambda b, i: (0, 0)),
      ],
      out_specs=pl.BlockSpec((1, SB, F), lambda b, i: (b, i, 0)),
      out_shape=jax.ShapeDtypeStruct((B, S, F), jnp.float32),
  )(x, w)


def _loss_body(n_ref, a_ref, x_ref, o_ref):
  i = pl.program_id(0)
  s = jnp.sum(jnp.abs(a_ref[...] - x_ref[...]))
  prev = jnp.where(i == 0, 0.0, o_ref[0, 0])
  tot = prev + s
  o_ref[0, 0] = jnp.where(i == pl.num_programs(0) - 1,
                          tot / n_ref[0], tot)


def _loss(agg_flat, xt_flat):
  N, F = agg_flat.shape
  RB = 1024
  n = jnp.full((1,), float(N * F), dtype=jnp.float32)
  out = pl.pallas_call(
      _loss_body,
      grid=(N // RB,),
      in_specs=[
          pl.BlockSpec(memory_space=pltpu.SMEM),
          pl.BlockSpec((RB, F), lambda i: (i, 0)),
          pl.BlockSpec((RB, F), lambda i: (i, 0)),
      ],
      out_specs=pl.BlockSpec(memory_space=pltpu.SMEM),
      out_shape=jax.ShapeDtypeStruct((1, 1), jnp.float32),
  )(n, agg_flat, xt_flat)
  return out[0, 0]


_B, _S, _F, _E = 4, 8192, 256, 32768
_NT = 16
_EPT = _E // _NT
_NPT = _S // _NT
_WIN = 128
_NWIN = _S // _WIN
_NPASS = _NWIN // _NT
_TRASH_E = _WIN
_BS = _B * _S
_PCH = 64
_PLSZ = _EPT + _NWIN * 16 + 16
_OSZ = _NPT + _NWIN * 16 + 16
_GPSZ = 52224
_GOSZ = 24576
_LANES = 16


def _sc_body(xt_hbm, nidx_hbm, eidx_hbm, agg_hbm,
             nbuf, ebuf2, ebuff, ones_v, zvec,
             cnt_v, best_v, mrg_v,
             nlall, elall, onall, oeall, ovall, onone,
             ctabp, ctabo, myrow, tabpv, tabov, exb,
             pnvec, pevec, pobig,
             gnb, geb, gidx, onb, oeb, ovb,
             rows, acc, ob, smem,
             hist_sp, stage_sp, tabp_sp, tabo_sp,
             gpn_sp, gpe_sp, gon_sp, goe_sp, gov_sp,
             sem_g, sem_o, sem_l):
  c = lax.axis_index("c")
  s = lax.axis_index("s")

  pltpu.sync_copy(nidx_hbm.at[pl.ds(s * _EPT, _EPT)], nbuf)
  pltpu.sync_copy(eidx_hbm.at[pl.ds(s * _EPT, _EPT)], ebuff)
  for j in range(16):
    pltpu.sync_copy(eidx_hbm.at[pl.ds(s * _EPT + j * 128, 128)], ebuf2.at[j])

  zf = jnp.zeros((_LANES,), jnp.float32)
  zi = jnp.zeros((_LANES,), jnp.int32)
  padn = zi
  pade = jnp.full((16,), _TRASH_E, jnp.int32)
  padbig = jnp.full((16,), 1 << 20, jnp.int32)

  def _init1(i, _):
    ones_v[pl.ds(i * 16, 16)] = jnp.ones((16,), jnp.float32)
    zvec[pl.ds(i * 16, 16)] = zf
    pnvec[pl.ds(i * 16, 16)] = padn
    pevec[pl.ds(i * 16, 16)] = pade
    pobig[pl.ds(i * 16, 16)] = padbig
    return 0
  lax.fori_loop(0, 32, _init1, 0)

  def _init2(i, _):
    best_v[pl.ds(i * 16, 16)] = jnp.full((16,), -1, jnp.int32)
    return 0
  lax.fori_loop(0, 512, _init2, 0)

  def _init3(i, _):
    ctabp[pl.ds(i * 16, 16)] = zi
    ctabo[pl.ds(i * 16, 16)] = zi
    return 0
  lax.fori_loop(0, 64, _init3, 0)

  def _init4(i, _):
    nlall[pl.ds(i * 16, 16)] = padn
    elall[pl.ds(i * 16, 16)] = pade
    return 0
  lax.fori_loop(0, _PLSZ // 16, _init4, 0)

  def _init5(i, _):
    onall[pl.ds(i * 16, 16)] = padbig
    oeall[pl.ds(i * 16, 16)] = pade
    ovall[pl.ds(i * 16, 16)] = zf
    return 0
  lax.fori_loop(0, _OSZ // 16, _init5, 0)

  def _init6(i, _):
    onone[pl.ds(i * 16, 16)] = padbig
    return 0
  lax.fori_loop(0, (_NPT + 16) // 16, _init6, 0)

  gp_stripe = _GPSZ // 16
  go_stripe = _GOSZ // 16
  for j in range(7):
    sz = min(512, gp_stripe - j * 512)
    if sz > 0:
      pltpu.sync_copy(pnvec.at[pl.ds(0, sz)],
                      gpn_sp.at[pl.ds(s * gp_stripe + j * 512, sz)])
      pltpu.sync_copy(pevec.at[pl.ds(0, sz)],
                      gpe_sp.at[pl.ds(s * gp_stripe + j * 512, sz)])
  for j in range(3):
    pltpu.sync_copy(pobig, gon_sp.at[pl.ds(s * go_stripe + j * 512, 512)])
    pltpu.sync_copy(pevec, goe_sp.at[pl.ds(s * go_stripe + j * 512, 512)])
    pltpu.sync_copy(zvec, gov_sp.at[pl.ds(s * go_stripe + j * 512, 512)])

  pltpu.sync_copy(zvec, hist_sp.at[pl.ds(s * 512, 512)])
  plsc.subcore_barrier()

  def _hist(j, _):
    pltpu.sync_copy(ones_v.at[pl.ds(0, 128)], hist_sp.at[ebuf2.at[j]],
                    add=True)
    return 0
  lax.fori_loop(0, 16, _hist, 0)
  plsc.subcore_barrier()

  pltpu.sync_copy(hist_sp, cnt_v)

  iot = lax.iota(jnp.int32, _LANES)

  def _pairs_a(g, cnts):
    n16 = nbuf[pl.ds(g * 16, 16)]
    e16 = ebuff[pl.ds(g * 16, 16)]
    cvals = plsc.load_gather(cnt_v, [e16])
    elig = cvals > 1.5
    cand = jnp.where(elig, e16, -1)
    comb = n16 * 16384 + (cand + 1)
    sk, _sv = plsc.sort_key_val(comb, comb)
    ns = lax.shift_right_logical(sk, 14)
    cs = (sk & 16383) - 1
    _, lastm = plsc.scan_count(ns)
    cur = plsc.load_gather(best_v, [ns])
    plsc.store_scatter(best_v, [ns], jnp.maximum(cur, cs), mask=lastm)
    w16 = lax.shift_right_logical(e16, 7)
    out = []
    for k in range(_NWIN):
      pcv = plsc.all_reduce_population_count(w16 == k)
      ctabp[pl.ds(k * 16, 16)] = ctabp[pl.ds(k * 16, 16)] + pcv
      out.append(cnts[k] + pcv[0])
    return tuple(out)

  cnt_p = lax.fori_loop(0, _EPT // 16, _pairs_a, (0,) * _NWIN)

  prow = []
  r0 = 0
  for k in range(_NWIN):
    prow.append(r0)
    r0 = r0 + (cnt_p[k] + 15) // 16

  def _pairs_b(g, offs):
    n16 = nbuf[pl.ds(g * 16, 16)]
    e16 = ebuff[pl.ds(g * 16, 16)]
    w16 = lax.shift_right_logical(e16, 7)
    eloc16 = e16 & (_WIN - 1)
    out = []
    for k in range(_NWIN):
      mk = w16 == k
      pck = plsc.all_reduce_population_count(mk)[0]
      plsc.store_compressed(nlall.at[pl.ds(offs[k], 16)], n16, mask=mk)
      plsc.store_compressed(elall.at[pl.ds(offs[k], 16)], eloc16, mask=mk)
      out.append(offs[k] + pck)
    return tuple(out)

  lax.fori_loop(0, _EPT // 16, _pairs_b, tuple(r * 16 for r in prow))

  pltpu.sync_copy(best_v, stage_sp.at[s])
  plsc.subcore_barrier()
  pltpu.sync_copy(stage_sp.at[:, pl.ds(s * _NPT, _NPT)], mrg_v)

  def _nodes_a(g, cnts):
    m = mrg_v[0, pl.ds(g * 16, 16)]
    for r in range(1, 16):
      m = jnp.maximum(m, mrg_v[r, pl.ds(g * 16, 16)])
    best_v[pl.ds(g * 16, 16)] = m
    mw = lax.shift_right_logical(jnp.maximum(m, 0), 7)
    valid = m >= 0
    out = []
    for k in range(_NWIN):
      pcv = plsc.all_reduce_population_count(valid & (mw == k))
      ctabo[pl.ds(k * 16, 16)] = ctabo[pl.ds(k * 16, 16)] + pcv
      out.append(cnts[k] + pcv[0])
    pcn = plsc.all_reduce_population_count(m < 0)[0]
    return tuple(out) + (cnts[_NWIN] + pcn,)

  cnt_o = lax.fori_loop(0, _NPT // 16, _nodes_a, (0,) * (_NWIN + 1))

  orow = []
  r0 = 0
  for k in range(_NWIN):
    orow.append(r0)
    r0 = r0 + (cnt_o[k] + 15) // 16

  def _nodes_b(g, offs):
    m = best_v[pl.ds(g * 16, 16)]
    nodeid = s * _NPT + g * 16 + iot
    cb = plsc.load_gather(cnt_v, [jnp.maximum(m, 0)])
    iv = 1.0 / jnp.maximum(cb, 1.0)
    valid = m >= 0
    mw = lax.shift_right_logical(jnp.maximum(m, 0), 7)
    mloc = jnp.maximum(m, 0) & (_WIN - 1)
    out = []
    for k in range(_NWIN):
      mk = valid & (mw == k)
      pck = plsc.all_reduce_population_count(mk)[0]
      plsc.store_compressed(onall.at[pl.ds(offs[k], 16)], nodeid, mask=mk)
      plsc.store_compressed(oeall.at[pl.ds(offs[k], 16)], mloc, mask=mk)
      plsc.store_compressed(ovall.at[pl.ds(offs[k], 16)], iv, mask=mk)
      out.append(offs[k] + pck)
    mn = m < 0
    pcn = plsc.all_reduce_population_count(mn)[0]
    plsc.store_compressed(onone.at[pl.ds(offs[_NWIN], 16)], nodeid, mask=mn)
    return tuple(out) + (offs[_NWIN] + pcn,)

  lax.fori_loop(0, _NPT // 16, _nodes_b,
                tuple(r * 16 for r in orow) + (0,))
  cnt_on = cnt_o[_NWIN]

  for k in range(_NWIN):
    smem[k] = prow[k]
    smem[_NWIN + k] = cnt_p[k]
    smem[2 * _NWIN + k] = orow[k]
    smem[3 * _NWIN + k] = cnt_o[k]

  for j in range(4):
    myrow[pl.ds(j * 16, 16)] = plsc.load_gather(ctabp, [(j * 16 + iot) * 16])
  pltpu.sync_copy(myrow, tabp_sp.at[pl.ds(s * 64, 64)])
  for j in range(4):
    myrow[pl.ds(j * 16, 16)] = plsc.load_gather(ctabo, [(j * 16 + iot) * 16])
  pltpu.sync_copy(myrow, tabo_sp.at[pl.ds(s * 64, 64)])
  plsc.subcore_barrier()
  pltpu.sync_copy(tabp_sp, tabpv)
  pltpu.sync_copy(tabo_sp, tabov)

  def _push(w, carry):
    gs, gso, ncp = carry
    wful = jnp.full((16,), w, jnp.int32)
    c16 = plsc.load_gather(tabpv, [iot * 64 + wful])
    c16p = (c16 + 15) & -16
    cum = plsc.cumsum(c16p)
    ex = cum - c16p
    exb[pl.ds(0, 16)] = ex
    myex = plsc.load_gather(exb, [jnp.full((16,), s, jnp.int32)])[0]
    tot = cum[15]
    gtot = (tot + 63) & -64
    smem[256 + w] = gs
    smem[320 + w] = gtot
    lstart = pl.multiple_of(smem[w] * 16, 16)
    mycnt = smem[_NWIN + w]
    myoff = pl.multiple_of(gs + myex, 16)

    def _fp(q, n):
      lo = pl.multiple_of(lstart + q * 16, 16)
      go = pl.multiple_of(myoff + q * 16, 16)
      pltpu.sync_copy(nlall.at[pl.ds(lo, 16)], gpn_sp.at[pl.ds(go, 16)])
      pltpu.sync_copy(elall.at[pl.ds(lo, 16)], gpe_sp.at[pl.ds(go, 16)])
      return n + 2
    ncp = lax.fori_loop(0, (mycnt + 15) // 16, _fp, ncp)

    o16 = plsc.load_gather(tabov, [iot * 64 + wful])
    o16p = (o16 + 15) & -16
    ocum = plsc.cumsum(o16p)
    oex = ocum - o16p
    exb[pl.ds(0, 16)] = oex
    myoex = plsc.load_gather(exb, [jnp.full((16,), s, jnp.int32)])[0]
    otot = ocum[15]
    smem[384 + w] = gso
    smem[448 + w] = otot
    olstart = pl.multiple_of(smem[2 * _NWIN + w] * 16, 16)
    myocnt = smem[3 * _NWIN + w]
    myooff = pl.multiple_of(gso + myoex, 16)

    def _fo(q, n):
      lo = pl.multiple_of(olstart + q * 16, 16)
      go = pl.multiple_of(myooff + q * 16, 16)
      pltpu.sync_copy(onall.at[pl.ds(lo, 16)], gon_sp.at[pl.ds(go, 16)])
      pltpu.sync_copy(oeall.at[pl.ds(lo, 16)], goe_sp.at[pl.ds(go, 16)])
      pltpu.sync_copy(ovall.at[pl.ds(lo, 16)], gov_sp.at[pl.ds(go, 16)])
      return n + 3
    ncp = lax.fori_loop(0, (myocnt + 15) // 16, _fo, ncp)
    return gs + gtot, gso + otot, ncp

  _, _, ncopies = lax.fori_loop(0, _NWIN, _push, (0, 0, 0))

  del ncopies
  plsc.subcore_barrier()

  def _none_pass(bi, _):
    bbase = (c * 2 + bi) * _S

    def _none(v, _v):
      nn = onone[pl.ds(v * 16, 16)]
      src = jnp.where(nn >= _S, 0, nn + bbase)
      dst = jnp.where(nn >= _S, _BS, nn + bbase)
      ob16 = ob.at[pl.ds(0, 16)]
      pltpu.async_copy(xt_hbm.at[src], ob16, sem_o).wait()
      pltpu.async_copy(ob16, agg_hbm.at[dst], sem_o).wait()
      return 0
    lax.fori_loop(0, (cnt_on + 15) // 16, _none, 0)
    return 0
  lax.fori_loop(0, 2, _none_pass, 0)

  def _bp(t, _):
    bi = lax.div(t, _NPASS)
    p = t - bi * _NPASS
    w = p * _NT + s
    bbase = (c * 2 + bi) * _S

    def _zero(i, _i):
      for k in range(16):
        acc[i, pl.ds(k * 16, 16)] = zf
      return 0
    lax.fori_loop(0, _WIN + 1, _zero, 0)

    gs = pl.multiple_of(smem[256 + w], 64)
    gc = smem[320 + w]

    def _chunk(q, _q):
      off = pl.multiple_of(gs + q * _PCH, 64)
      pltpu.sync_copy(gpn_sp.at[pl.ds(off, _PCH)], gnb)
      pltpu.sync_copy(gpe_sp.at[pl.ds(off, _PCH)], geb)
      for m in range(_PCH // 16):
        gidx[pl.ds(m * 16, 16)] = gnb[pl.ds(m * 16, 16)] + bbase
      pltpu.async_copy(xt_hbm.at[gidx], rows, sem_g).wait()

      def _addrow(rr, _r):
        el16 = geb[pl.ds(rr * 16, 16)]
        for r in range(16):
          e = el16[r]
          for cg in range(16):
            plsc.addupdate(acc.at[e, pl.ds(cg * 16, 16)],
                           rows[rr * 16 + r, pl.ds(cg * 16, 16)])
        return 0
      lax.fori_loop(0, _PCH // 16, _addrow, 0)
      return 0
    lax.fori_loop(0, (gc + _PCH - 1) // _PCH, _chunk, 0)

    gso = pl.multiple_of(smem[384 + w], 16)
    gco = smem[448 + w]

    def _ochunk(v, _v):
      off = pl.multiple_of(gso + v * 16, 16)
      pltpu.async_copy(gon_sp.at[pl.ds(off, 16)], onb, sem_o)
      pltpu.async_copy(goe_sp.at[pl.ds(off, 16)], oeb, sem_o)
      pltpu.async_copy(gov_sp.at[pl.ds(off, 16)], ovb, sem_o)
      pltpu.make_async_copy(gon_sp.at[pl.ds(off, 16)], onb, sem_o).wait()
      pltpu.make_async_copy(goe_sp.at[pl.ds(off, 16)], oeb, sem_o).wait()
      pltpu.make_async_copy(gov_sp.at[pl.ds(off, 16)], ovb, sem_o).wait()
      nn = onb[...]
      el16 = oeb[...]
      iv16 = ovb[...]
      dst = jnp.where(nn >= _S, _BS, nn + bbase)
      ob16 = ob.at[pl.ds(0, 16)]
      for r in range(16):
        e = el16[r]
        sc = iv16[r]
        for cg in range(16):
          ob[r, pl.ds(cg * 16, 16)] = acc[e, pl.ds(cg * 16, 16)] * sc
      pltpu.async_copy(ob16, agg_hbm.at[dst], sem_o).wait()
      return 0
    lax.fori_loop(0, (gco + 15) // 16, _ochunk, 0)
    return 0

  lax.fori_loop(0, 2 * _NPASS, _bp, 0)


def _sc_gather_scatter(xt_flat, node_idx, edge_idx):
  mesh = plsc.VectorSubcoreMesh(core_axis_name="c", subcore_axis_name="s")
  f = pl.kernel(
      _sc_body,
      out_type=jax.ShapeDtypeStruct((_BS + 64, _F), jnp.float32),
      mesh=mesh,
      compiler_params=pltpu.CompilerParams(needs_layout_passes=False),
      scratch_types=[
          pltpu.VMEM((_EPT,), jnp.int32),
          pltpu.VMEM((16, 128), jnp.int32),
          pltpu.VMEM((_EPT,), jnp.int32),
          pltpu.VMEM((512,), jnp.float32),
          pltpu.VMEM((512,), jnp.float32),
          pltpu.VMEM((_S,), jnp.float32),
          pltpu.VMEM((_S,), jnp.int32),
          pltpu.VMEM((16, _NPT), jnp.int32),
          pltpu.VMEM((_PLSZ,), jnp.int32),
          pltpu.VMEM((_PLSZ,), jnp.int32),
          pltpu.VMEM((_OSZ,), jnp.int32),
          pltpu.VMEM((_OSZ,), jnp.int32),
          pltpu.VMEM((_OSZ,), jnp.float32),
          pltpu.VMEM((_NPT + 16,), jnp.int32),
          pltpu.VMEM((1024,), jnp.int32),
          pltpu.VMEM((1024,), jnp.int32),
          pltpu.VMEM((64,), jnp.int32),
          pltpu.VMEM((1024,), jnp.int32),
          pltpu.VMEM((1024,), jnp.int32),
          pltpu.VMEM((32,), jnp.int32),
          pltpu.VMEM((512,), jnp.int32),
          pltpu.VMEM((512,), jnp.int32),
          pltpu.VMEM((512,), jnp.int32),
          pltpu.VMEM((_PCH,), jnp.int32),
          pltpu.VMEM((_PCH,), jnp.int32),
          pltpu.VMEM((_PCH,), jnp.int32),
          pltpu.VMEM((16,), jnp.int32),
          pltpu.VMEM((16,), jnp.int32),
          pltpu.VMEM((16,), jnp.float32),
          pltpu.VMEM((_PCH, _F), jnp.float32),
          pltpu.VMEM((_WIN + 1, _F), jnp.float32),
          pltpu.VMEM((16, _F), jnp.float32),
          pltpu.SMEM((544,), jnp.int32),
          pltpu.VMEM_SHARED((_S,), jnp.float32),
          pltpu.VMEM_SHARED((16, _S), jnp.int32),
          pltpu.VMEM_SHARED((1024,), jnp.int32),
          pltpu.VMEM_SHARED((1024,), jnp.int32),
          pltpu.VMEM_SHARED((_GPSZ,), jnp.int32),
          pltpu.VMEM_SHARED((_GPSZ,), jnp.int32),
          pltpu.VMEM_SHARED((_GOSZ,), jnp.int32),
          pltpu.VMEM_SHARED((_GOSZ,), jnp.int32),
          pltpu.VMEM_SHARED((_GOSZ,), jnp.float32),
          pltpu.SemaphoreType.DMA,
          pltpu.SemaphoreType.DMA,
          pltpu.SemaphoreType.DMA,
      ],
  )
  return f(xt_flat, node_idx, edge_idx)


def kernel(x, hyperedge_index, weight):
  xt = _matmul(x, weight)
  xt_flat = xt.reshape(_BS, _F)
  node_idx = hyperedge_index[0]
  edge_idx = hyperedge_index[1]
  agg_pad = _sc_gather_scatter(xt_flat, node_idx, edge_idx)
  agg = agg_pad[:_BS].reshape(_B, _S, _F)
  loss = _loss(agg_pad[:_BS], xt_flat)
  return agg, loss


# speedup vs baseline: 2.5172x; 1.0289x over previous
import jax
import jax.numpy as jnp
from jax import lax
from jax.experimental import pallas as pl
from jax.experimental.pallas import tpu as pltpu
from jax.experimental.pallas import tpu_sc as plsc


def _mm_body(x_ref, w_ref, o_ref):
  o_ref[0] = jnp.dot(x_ref[0], w_ref[...],
                     preferred_element_type=jnp.float32)


def _matmul(x, w):
  B, S, FIN = x.shape
  F = w.shape[1]
  SB = 512
  return pl.pallas_call(
      _mm_body,
      grid=(B, S // SB),
      in_specs=[
          pl.BlockSpec((1, SB, FIN), lambda b, i: (b, i, 0)),
          pl.BlockSpec((FIN, F), lambda b, i: (0, 0)),
      ],
      out_specs=pl.BlockSpec((1, SB, F), lambda b, i: (b, i, 0)),
      out_shape=jax.ShapeDtypeStruct((B, S, F), jnp.float32),
  )(x, w)


def _loss_body(n_ref, a_ref, x_ref, o_ref):
  i = pl.program_id(0)
  s = jnp.sum(jnp.abs(a_ref[...] - x_ref[...]))
  prev = jnp.where(i == 0, 0.0, o_ref[0, 0])
  tot = prev + s
  o_ref[0, 0] = jnp.where(i == pl.num_programs(0) - 1,
                          tot / n_ref[0], tot)


def _loss(agg_flat, xt_flat):
  N, F = agg_flat.shape
  RB = 1024
  n = jnp.full((1,), float(N * F), dtype=jnp.float32)
  out = pl.pallas_call(
      _loss_body,
      grid=(N // RB,),
      in_specs=[
          pl.BlockSpec(memory_space=pltpu.SMEM),
          pl.BlockSpec((RB, F), lambda i: (i, 0)),
          pl.BlockSpec((RB, F), lambda i: (i, 0)),
      ],
      out_specs=pl.BlockSpec(memory_space=pltpu.SMEM),
      out_shape=jax.ShapeDtypeStruct((1, 1), jnp.float32),
  )(n, agg_flat, xt_flat)
  return out[0, 0]


_B, _S, _F, _E = 4, 8192, 256, 32768
_NT = 16
_EPT = _E // _NT
_NPT = _S // _NT
_WIN = 128
_NWIN = _S // _WIN
_NPASS = _NWIN // _NT
_TRASH_E = _WIN
_BS = _B * _S
_PCH = 64
_PLSZ = _EPT + _NWIN * 16 + 16
_OSZ = _NPT + _NWIN * 16 + 16
_GPSZ = 52224
_GOSZ = 24576
_LANES = 16


def _sc_body(xt_hbm, nidx_hbm, eidx_hbm, agg_hbm,
             nbuf, ebuf2, ebuff, ones_v, zvec,
             cnt_v, best_v, mrg_v,
             nlall, elall, onall, oeall, ovall, onone,
             cnts_v, offs_v, ocnts_v, ooffs_v, tabpv, tabov, exb,
             pnvec, pevec, pobig,
             gnb, geb, gidx, onb, oeb, ovb,
             rows, acc, ob, smem,
             hist_sp, stage_sp, tabp_sp, tabo_sp,
             gpn_sp, gpe_sp, gon_sp, goe_sp, gov_sp,
             sem_g, sem_o, sem_l):
  c = lax.axis_index("c")
  s = lax.axis_index("s")

  pltpu.sync_copy(nidx_hbm.at[pl.ds(s * _EPT, _EPT)], nbuf)
  pltpu.sync_copy(eidx_hbm.at[pl.ds(s * _EPT, _EPT)], ebuff)
  for j in range(16):
    pltpu.sync_copy(eidx_hbm.at[pl.ds(s * _EPT + j * 128, 128)], ebuf2.at[j])

  zf = jnp.zeros((_LANES,), jnp.float32)
  zi = jnp.zeros((_LANES,), jnp.int32)
  padn = zi
  pade = jnp.full((16,), _TRASH_E, jnp.int32)
  padbig = jnp.full((16,), 1 << 20, jnp.int32)

  def _init1(i, _):
    ones_v[pl.ds(i * 16, 16)] = jnp.ones((16,), jnp.float32)
    zvec[pl.ds(i * 16, 16)] = zf
    pnvec[pl.ds(i * 16, 16)] = padn
    pevec[pl.ds(i * 16, 16)] = pade
    pobig[pl.ds(i * 16, 16)] = padbig
    return 0
  lax.fori_loop(0, 32, _init1, 0)

  def _init2(i, _):
    best_v[pl.ds(i * 16, 16)] = jnp.full((16,), -1, jnp.int32)
    return 0
  lax.fori_loop(0, 512, _init2, 0)

  def _init3(i, _):
    cnts_v[pl.ds(i * 16, 16)] = zi
    offs_v[pl.ds(i * 16, 16)] = zi
    ocnts_v[pl.ds(i * 16, 16)] = zi
    ooffs_v[pl.ds(i * 16, 16)] = zi
    return 0
  lax.fori_loop(0, 5, _init3, 0)

  def _init4(i, _):
    nlall[pl.ds(i * 16, 16)] = padn
    elall[pl.ds(i * 16, 16)] = pade
    return 0
  lax.fori_loop(0, _PLSZ // 16, _init4, 0)

  def _init5(i, _):
    onall[pl.ds(i * 16, 16)] = padbig
    oeall[pl.ds(i * 16, 16)] = pade
    ovall[pl.ds(i * 16, 16)] = zf
    return 0
  lax.fori_loop(0, _OSZ // 16, _init5, 0)

  def _init6(i, _):
    onone[pl.ds(i * 16, 16)] = padbig
    return 0
  lax.fori_loop(0, (_NPT + 16) // 16, _init6, 0)

  gp_stripe = _GPSZ // 16
  go_stripe = _GOSZ // 16
  for j in range(7):
    sz = min(512, gp_stripe - j * 512)
    if sz > 0:
      pltpu.sync_copy(pnvec.at[pl.ds(0, sz)],
                      gpn_sp.at[pl.ds(s * gp_stripe + j * 512, sz)])
      pltpu.sync_copy(pevec.at[pl.ds(0, sz)],
                      gpe_sp.at[pl.ds(s * gp_stripe + j * 512, sz)])
  for j in range(3):
    pltpu.sync_copy(pobig, gon_sp.at[pl.ds(s * go_stripe + j * 512, 512)])
    pltpu.sync_copy(pevec, goe_sp.at[pl.ds(s * go_stripe + j * 512, 512)])
    pltpu.sync_copy(zvec, gov_sp.at[pl.ds(s * go_stripe + j * 512, 512)])

  pltpu.sync_copy(zvec, hist_sp.at[pl.ds(s * 512, 512)])
  plsc.subcore_barrier()

  def _hist(j, _):
    pltpu.sync_copy(ones_v.at[pl.ds(0, 128)], hist_sp.at[ebuf2.at[j]],
                    add=True)
    return 0
  lax.fori_loop(0, 16, _hist, 0)
  plsc.subcore_barrier()

  pltpu.sync_copy(hist_sp, cnt_v)

  iot = lax.iota(jnp.int32, _LANES)
  rz, _ = plsc.scan_count(zi)
  bias = rz[15] - 15

  def _pairs_a(g, _):
    n16 = nbuf[pl.ds(g * 16, 16)]
    e16 = ebuff[pl.ds(g * 16, 16)]
    cvals = plsc.load_gather(cnt_v, [e16])
    elig = cvals > 1.5
    cand = jnp.where(elig, e16, -1)
    comb = n16 * 16384 + (cand + 1)
    sk, _sv = plsc.sort_key_val(comb, comb)
    ns = lax.shift_right_logical(sk, 14)
    cs = (sk & 16383) - 1
    _, lastm = plsc.scan_count(ns)
    cur = plsc.load_gather(best_v, [ns])
    plsc.store_scatter(best_v, [ns], jnp.maximum(cur, cs), mask=lastm)
    w16 = lax.shift_right_logical(e16, 7)
    rank, wl = plsc.scan_count(w16)
    curw = plsc.load_gather(cnts_v, [w16])
    plsc.store_scatter(cnts_v, [w16], curw + (rank - bias) + 1, mask=wl)
    return 0

  lax.fori_loop(0, _EPT // 16, _pairs_a, 0)

  carry = 0
  for j in range(4):
    c16 = cnts_v[pl.ds(j * 16, 16)]
    v = (c16 + 15) & -16
    cum = plsc.cumsum(v) + carry
    st = cum - v
    offs_v[pl.ds(j * 16, 16)] = st
    carry = cum[15]
    for i in range(16):
      smem[j * 16 + i] = st[i]
      smem[_NWIN + j * 16 + i] = c16[i]

  def _pairs_b(g, _):
    n16 = nbuf[pl.ds(g * 16, 16)]
    e16 = ebuff[pl.ds(g * 16, 16)]
    w16 = lax.shift_right_logical(e16, 7)
    eloc16 = e16 & (_WIN - 1)
    rank, wl = plsc.scan_count(w16)
    base = plsc.load_gather(offs_v, [w16])
    dest = base + (rank - bias)
    plsc.store_scatter(nlall, [dest], n16)
    plsc.store_scatter(elall, [dest], eloc16)
    plsc.store_scatter(offs_v, [w16], dest + 1, mask=wl)
    return 0

  lax.fori_loop(0, _EPT // 16, _pairs_b, 0)

  pltpu.sync_copy(best_v, stage_sp.at[s])
  plsc.subcore_barrier()
  pltpu.sync_copy(stage_sp.at[:, pl.ds(s * _NPT, _NPT)], mrg_v)

  def _nodes_a(g, _):
    m = mrg_v[0, pl.ds(g * 16, 16)]
    for r in range(1, 16):
      m = jnp.maximum(m, mrg_v[r, pl.ds(g * 16, 16)])
    best_v[pl.ds(g * 16, 16)] = m
    mw = lax.shift_right_logical(jnp.maximum(m, 0), 7)
    valid = m >= 0
    rank, wl = plsc.scan_count(mw, mask=valid)
    curw = plsc.load_gather(ocnts_v, [mw])
    plsc.store_scatter(ocnts_v, [mw], curw + (rank - bias) + 1, mask=wl)
    return 0

  lax.fori_loop(0, _NPT // 16, _nodes_a, 0)

  carry = 0
  for j in range(4):
    c16 = ocnts_v[pl.ds(j * 16, 16)]
    v = (c16 + 15) & -16
    cum = plsc.cumsum(v) + carry
    st = cum - v
    ooffs_v[pl.ds(j * 16, 16)] = st
    carry = cum[15]
    for i in range(16):
      smem[2 * _NWIN + j * 16 + i] = st[i]
      smem[3 * _NWIN + j * 16 + i] = c16[i]

  def _nodes_b(g, onoff):
    m = best_v[pl.ds(g * 16, 16)]
    nodeid = s * _NPT + g * 16 + iot
    cb = plsc.load_gather(cnt_v, [jnp.maximum(m, 0)])
    iv = 1.0 / jnp.maximum(cb, 1.0)
    valid = m >= 0
    mw = lax.shift_right_logical(jnp.maximum(m, 0), 7)
    mloc = jnp.maximum(m, 0) & (_WIN - 1)
    rank, wl = plsc.scan_count(mw, mask=valid)
    base = plsc.load_gather(ooffs_v, [mw])
    dest = base + (rank - bias)
    plsc.store_scatter(onall, [dest], nodeid, mask=valid)
    plsc.store_scatter(oeall, [dest], mloc, mask=valid)
    plsc.store_scatter(ovall, [dest], iv, mask=valid)
    plsc.store_scatter(ooffs_v, [mw], dest + 1, mask=wl)
    mn = m < 0
    pcn = plsc.all_reduce_population_count(mn)[0]
    plsc.store_compressed(onone.at[pl.ds(onoff, 16)], nodeid, mask=mn)
    return onoff + pcn

  cnt_on = lax.fori_loop(0, _NPT // 16, _nodes_b, 0)

  pltpu.sync_copy(cnts_v.at[pl.ds(0, 64)], tabp_sp.at[pl.ds(s * 64, 64)])
  pltpu.sync_copy(ocnts_v.at[pl.ds(0, 64)], tabo_sp.at[pl.ds(s * 64, 64)])
  plsc.subcore_barrier()
  pltpu.sync_copy(tabp_sp, tabpv)
  pltpu.sync_copy(tabo_sp, tabov)

  def _push(w, carry):
    gs, gso, ncp = carry
    wful = jnp.full((16,), w, jnp.int32)
    c16 = plsc.load_gather(tabpv, [iot * 64 + wful])
    c16p = (c16 + 15) & -16
    cum = plsc.cumsum(c16p)
    ex = cum - c16p
    exb[pl.ds(0, 16)] = ex
    myex = plsc.load_gather(exb, [jnp.full((16,), s, jnp.int32)])[0]
    tot = cum[15]
    gtot = (tot + 63) & -64
    smem[256 + w] = gs
    smem[320 + w] = gtot
    lstart = pl.multiple_of(smem[w], 16)
    mycnt = smem[_NWIN + w]
    myoff = pl.multiple_of(gs + myex, 16)

    def _fp(q, n):
      lo = pl.multiple_of(lstart + q * 16, 16)
      go = pl.multiple_of(myoff + q * 16, 16)
      pltpu.sync_copy(nlall.at[pl.ds(lo, 16)], gpn_sp.at[pl.ds(go, 16)])
      pltpu.sync_copy(elall.at[pl.ds(lo, 16)], gpe_sp.at[pl.ds(go, 16)])
      return n + 2
    ncp = lax.fori_loop(0, (mycnt + 15) // 16, _fp, ncp)

    o16 = plsc.load_gather(tabov, [iot * 64 + wful])
    o16p = (o16 + 15) & -16
    ocum = plsc.cumsum(o16p)
    oex = ocum - o16p
    exb[pl.ds(0, 16)] = oex
    myoex = plsc.load_gather(exb, [jnp.full((16,), s, jnp.int32)])[0]
    otot = ocum[15]
    smem[384 + w] = gso
    smem[448 + w] = otot
    olstart = pl.multiple_of(smem[2 * _NWIN + w], 16)
    myocnt = smem[3 * _NWIN + w]
    myooff = pl.multiple_of(gso + myoex, 16)

    def _fo(q, n):
      lo = pl.multiple_of(olstart + q * 16, 16)
      go = pl.multiple_of(myooff + q * 16, 16)
      pltpu.sync_copy(onall.at[pl.ds(lo, 16)], gon_sp.at[pl.ds(go, 16)])
      pltpu.sync_copy(oeall.at[pl.ds(lo, 16)], goe_sp.at[pl.ds(go, 16)])
      pltpu.sync_copy(ovall.at[pl.ds(lo, 16)], gov_sp.at[pl.ds(go, 16)])
      return n + 3
    ncp = lax.fori_loop(0, (myocnt + 15) // 16, _fo, ncp)
    return gs + gtot, gso + otot, ncp

  _, _, ncopies = lax.fori_loop(0, _NWIN, _push, (0, 0, 0))

  del ncopies
  plsc.subcore_barrier()

  def _none_pass(bi, _):
    bbase = (c * 2 + bi) * _S

    def _none(v, _v):
      nn = onone[pl.ds(v * 16, 16)]
      src = jnp.where(nn >= _S, 0, nn + bbase)
      dst = jnp.where(nn >= _S, _BS, nn + bbase)
      ob16 = ob.at[pl.ds(0, 16)]
      pltpu.async_copy(xt_hbm.at[src], ob16, sem_o).wait()
      pltpu.async_copy(ob16, agg_hbm.at[dst], sem_o).wait()
      return 0
    lax.fori_loop(0, (cnt_on + 15) // 16, _none, 0)
    return 0
  lax.fori_loop(0, 2, _none_pass, 0)

  def _bp(t, _):
    bi = lax.div(t, _NPASS)
    p = t - bi * _NPASS
    w = p * _NT + s
    bbase = (c * 2 + bi) * _S

    def _zero(i, _i):
      for k in range(16):
        acc[i, pl.ds(k * 16, 16)] = zf
      return 0
    lax.fori_loop(0, _WIN + 1, _zero, 0)

    gs = pl.multiple_of(smem[256 + w], 64)
    gc = smem[320 + w]

    def _chunk(q, _q):
      off = pl.multiple_of(gs + q * _PCH, 64)
      pltpu.sync_copy(gpn_sp.at[pl.ds(off, _PCH)], gnb)
      pltpu.sync_copy(gpe_sp.at[pl.ds(off, _PCH)], geb)
      for m in range(_PCH // 16):
        gidx[pl.ds(m * 16, 16)] = gnb[pl.ds(m * 16, 16)] + bbase
      pltpu.async_copy(xt_hbm.at[gidx], rows, sem_g).wait()

      def _addrow(rr, _r):
        el16 = geb[pl.ds(rr * 16, 16)]
        for r in range(16):
          e = el16[r]
          for cg in range(16):
            plsc.addupdate(acc.at[e, pl.ds(cg * 16, 16)],
                           rows[rr * 16 + r, pl.ds(cg * 16, 16)])
        return 0
      lax.fori_loop(0, _PCH // 16, _addrow, 0)
      return 0
    lax.fori_loop(0, (gc + _PCH - 1) // _PCH, _chunk, 0)

    gso = pl.multiple_of(smem[384 + w], 16)
    gco = smem[448 + w]

    def _ochunk(v, _v):
      off = pl.multiple_of(gso + v * 16, 16)
      pltpu.async_copy(gon_sp.at[pl.ds(off, 16)], onb, sem_o)
      pltpu.async_copy(goe_sp.at[pl.ds(off, 16)], oeb, sem_o)
      pltpu.async_copy(gov_sp.at[pl.ds(off, 16)], ovb, sem_o)
      pltpu.make_async_copy(gon_sp.at[pl.ds(off, 16)], onb, sem_o).wait()
      pltpu.make_async_copy(goe_sp.at[pl.ds(off, 16)], oeb, sem_o).wait()
      pltpu.make_async_copy(gov_sp.at[pl.ds(off, 16)], ovb, sem_o).wait()
      nn = onb[...]
      el16 = oeb[...]
      iv16 = ovb[...]
      dst = jnp.where(nn >= _S, _BS, nn + bbase)
      ob16 = ob.at[pl.ds(0, 16)]
      for r in range(16):
        e = el16[r]
        sc = iv16[r]
        for cg in range(16):
          ob[r, pl.ds(cg * 16, 16)] = acc[e, pl.ds(cg * 16, 16)] * sc
      pltpu.async_copy(ob16, agg_hbm.at[dst], sem_o).wait()
      return 0
    lax.fori_loop(0, (gco + 15) // 16, _ochunk, 0)
    return 0

  lax.fori_loop(0, 2 * _NPASS, _bp, 0)


def _sc_gather_scatter(xt_flat, node_idx, edge_idx):
  mesh = plsc.VectorSubcoreMesh(core_axis_name="c", subcore_axis_name="s")
  f = pl.kernel(
      _sc_body,
      out_type=jax.ShapeDtypeStruct((_BS + 64, _F), jnp.float32),
      mesh=mesh,
      compiler_params=pltpu.CompilerParams(needs_layout_passes=False),
      scratch_types=[
          pltpu.VMEM((_EPT,), jnp.int32),
          pltpu.VMEM((16, 128), jnp.int32),
          pltpu.VMEM((_EPT,), jnp.int32),
          pltpu.VMEM((512,), jnp.float32),
          pltpu.VMEM((512,), jnp.float32),
          pltpu.VMEM((_S,), jnp.float32),
          pltpu.VMEM((_S,), jnp.int32),
          pltpu.VMEM((16, _NPT), jnp.int32),
          pltpu.VMEM((_PLSZ,), jnp.int32),
          pltpu.VMEM((_PLSZ,), jnp.int32),
          pltpu.VMEM((_OSZ,), jnp.int32),
          pltpu.VMEM((_OSZ,), jnp.int32),
          pltpu.VMEM((_OSZ,), jnp.float32),
          pltpu.VMEM((_NPT + 16,), jnp.int32),
          pltpu.VMEM((80,), jnp.int32),
          pltpu.VMEM((80,), jnp.int32),
          pltpu.VMEM((80,), jnp.int32),
          pltpu.VMEM((80,), jnp.int32),
          pltpu.VMEM((1024,), jnp.int32),
          pltpu.VMEM((1024,), jnp.int32),
          pltpu.VMEM((32,), jnp.int32),
          pltpu.VMEM((512,), jnp.int32),
          pltpu.VMEM((512,), jnp.int32),
          pltpu.VMEM((512,), jnp.int32),
          pltpu.VMEM((_PCH,), jnp.int32),
          pltpu.VMEM((_PCH,), jnp.int32),
          pltpu.VMEM((_PCH,), jnp.int32),
          pltpu.VMEM((16,), jnp.int32),
          pltpu.VMEM((16,), jnp.int32),
          pltpu.VMEM((16,), jnp.float32),
          pltpu.VMEM((_PCH, _F), jnp.float32),
          pltpu.VMEM((_WIN + 1, _F), jnp.float32),
          pltpu.VMEM((16, _F), jnp.float32),
          pltpu.SMEM((544,), jnp.int32),
          pltpu.VMEM_SHARED((_S,), jnp.float32),
          pltpu.VMEM_SHARED((16, _S), jnp.int32),
          pltpu.VMEM_SHARED((1024,), jnp.int32),
          pltpu.VMEM_SHARED((1024,), jnp.int32),
          pltpu.VMEM_SHARED((_GPSZ,), jnp.int32),
          pltpu.VMEM_SHARED((_GPSZ,), jnp.int32),
          pltpu.VMEM_SHARED((_GOSZ,), jnp.int32),
          pltpu.VMEM_SHARED((_GOSZ,), jnp.int32),
          pltpu.VMEM_SHARED((_GOSZ,), jnp.float32),
          pltpu.SemaphoreType.DMA,
          pltpu.SemaphoreType.DMA,
          pltpu.SemaphoreType.DMA,
      ],
  )
  return f(xt_flat, node_idx, edge_idx)


def kernel(x, hyperedge_index, weight):
  xt = _matmul(x, weight)
  xt_flat = xt.reshape(_BS, _F)
  node_idx = hyperedge_index[0]
  edge_idx = hyperedge_index[1]
  agg_pad = _sc_gather_scatter(xt_flat, node_idx, edge_idx)
  agg = agg_pad[:_BS].reshape(_B, _S, _F)
  loss = _loss(agg_pad[:_BS], xt_flat)
  return agg, loss


# speedup vs baseline: 2.5465x; 1.0116x over previous
import jax
import jax.numpy as jnp
from jax import lax
from jax.experimental import pallas as pl
from jax.experimental.pallas import tpu as pltpu
from jax.experimental.pallas import tpu_sc as plsc


def _mm_body(x_ref, w_ref, o_ref):
  o_ref[0] = jnp.dot(x_ref[0], w_ref[...],
                     preferred_element_type=jnp.float32)


def _matmul(x, w):
  B, S, FIN = x.shape
  F = w.shape[1]
  SB = 512
  return pl.pallas_call(
      _mm_body,
      grid=(B, S // SB),
      in_specs=[
          pl.BlockSpec((1, SB, FIN), lambda b, i: (b, i, 0)),
          pl.BlockSpec((FIN, F), lambda b, i: (0, 0)),
      ],
      out_specs=pl.BlockSpec((1, SB, F), lambda b, i: (b, i, 0)),
      out_shape=jax.ShapeDtypeStruct((B, S, F), jnp.float32),
  )(x, w)


def _loss_body(n_ref, a_ref, x_ref, o_ref):
  i = pl.program_id(0)
  s = jnp.sum(jnp.abs(a_ref[...] - x_ref[...]))
  prev = jnp.where(i == 0, 0.0, o_ref[0, 0])
  tot = prev + s
  o_ref[0, 0] = jnp.where(i == pl.num_programs(0) - 1,
                          tot / n_ref[0], tot)


def _loss(agg_flat, xt_flat):
  N, F = agg_flat.shape
  RB = 1024
  n = jnp.full((1,), float(N * F), dtype=jnp.float32)
  out = pl.pallas_call(
      _loss_body,
      grid=(N // RB,),
      in_specs=[
          pl.BlockSpec(memory_space=pltpu.SMEM),
          pl.BlockSpec((RB, F), lambda i: (i, 0)),
          pl.BlockSpec((RB, F), lambda i: (i, 0)),
      ],
      out_specs=pl.BlockSpec(memory_space=pltpu.SMEM),
      out_shape=jax.ShapeDtypeStruct((1, 1), jnp.float32),
  )(n, agg_flat, xt_flat)
  return out[0, 0]


_B, _S, _F, _E = 4, 8192, 256, 32768
_NT = 16
_EPT = _E // _NT
_NPT = _S // _NT
_WIN = 128
_NWIN = _S // _WIN
_NPASS = _NWIN // _NT
_TRASH_E = _WIN
_BS = _B * _S
_PCH = 64
_PLSZ = _EPT + _NWIN * 16 + 16
_OSZ = _NPT + _NWIN * 16 + 16
_GPSZ = 52224
_GOSZ = 24576
_LANES = 16


def _sc_body(xt_hbm, nidx_hbm, eidx_hbm, agg_hbm,
             nbuf, ebuf2, ebuff, ones_v, zvec,
             cnt_v, best_v, mrg_v,
             nlall, elall, onall, oeall, ovall, onone,
             cnts_v, offs_v, ocnts_v, ooffs_v, tabpv, tabov, exb,
             pnvec, pevec, pobig,
             gnb, geb, gidx, onb, oeb, ovb,
             rows, acc, ob, smem,
             hist_sp, stage_sp, tabp_sp, tabo_sp,
             gpn_sp, gpe_sp, gon_sp, goe_sp, gov_sp,
             sem_g, sem_o, sem_l):
  c = lax.axis_index("c")
  s = lax.axis_index("s")

  pltpu.sync_copy(nidx_hbm.at[pl.ds(s * _EPT, _EPT)], nbuf)
  pltpu.sync_copy(eidx_hbm.at[pl.ds(s * _EPT, _EPT)], ebuff)
  for j in range(16):
    pltpu.sync_copy(eidx_hbm.at[pl.ds(s * _EPT + j * 128, 128)], ebuf2.at[j])

  zf = jnp.zeros((_LANES,), jnp.float32)
  zi = jnp.zeros((_LANES,), jnp.int32)
  padn = zi
  pade = jnp.full((16,), _TRASH_E, jnp.int32)
  padbig = jnp.full((16,), 1 << 20, jnp.int32)

  def _init1(i, _):
    ones_v[pl.ds(i * 16, 16)] = jnp.ones((16,), jnp.float32)
    zvec[pl.ds(i * 16, 16)] = zf
    pnvec[pl.ds(i * 16, 16)] = padn
    pevec[pl.ds(i * 16, 16)] = pade
    pobig[pl.ds(i * 16, 16)] = padbig
    return 0
  lax.fori_loop(0, 32, _init1, 0)

  def _init2(i, _):
    best_v[pl.ds(i * 16, 16)] = jnp.full((16,), -1, jnp.int32)
    return 0
  lax.fori_loop(0, 512, _init2, 0)

  def _init3(i, _):
    cnts_v[pl.ds(i * 16, 16)] = zi
    offs_v[pl.ds(i * 16, 16)] = zi
    ocnts_v[pl.ds(i * 16, 16)] = zi
    ooffs_v[pl.ds(i * 16, 16)] = zi
    return 0
  lax.fori_loop(0, 5, _init3, 0)

  def _init4(i, _):
    nlall[pl.ds(i * 16, 16)] = padn
    elall[pl.ds(i * 16, 16)] = pade
    return 0
  lax.fori_loop(0, _PLSZ // 16, _init4, 0)

  def _init5(i, _):
    onall[pl.ds(i * 16, 16)] = padbig
    oeall[pl.ds(i * 16, 16)] = pade
    ovall[pl.ds(i * 16, 16)] = zf
    return 0
  lax.fori_loop(0, _OSZ // 16, _init5, 0)

  def _init6(i, _):
    onone[pl.ds(i * 16, 16)] = padbig
    return 0
  lax.fori_loop(0, (_NPT + 16) // 16, _init6, 0)

  gp_stripe = _GPSZ // 16
  go_stripe = _GOSZ // 16
  for j in range(7):
    sz = min(512, gp_stripe - j * 512)
    if sz > 0:
      pltpu.sync_copy(pnvec.at[pl.ds(0, sz)],
                      gpn_sp.at[pl.ds(s * gp_stripe + j * 512, sz)])
      pltpu.sync_copy(pevec.at[pl.ds(0, sz)],
                      gpe_sp.at[pl.ds(s * gp_stripe + j * 512, sz)])
  for j in range(3):
    pltpu.sync_copy(pobig, gon_sp.at[pl.ds(s * go_stripe + j * 512, 512)])
    pltpu.sync_copy(pevec, goe_sp.at[pl.ds(s * go_stripe + j * 512, 512)])
    pltpu.sync_copy(zvec, gov_sp.at[pl.ds(s * go_stripe + j * 512, 512)])

  pltpu.sync_copy(zvec, hist_sp.at[pl.ds(s * 512, 512)])
  plsc.subcore_barrier()

  def _hist(j, _):
    pltpu.sync_copy(ones_v.at[pl.ds(0, 128)], hist_sp.at[ebuf2.at[j]],
                    add=True)
    return 0
  lax.fori_loop(0, 16, _hist, 0)
  plsc.subcore_barrier()

  pltpu.sync_copy(hist_sp, cnt_v)

  iot = lax.iota(jnp.int32, _LANES)
  rz, _ = plsc.scan_count(zi)
  bias = rz[15] - 15

  def _pairs_a(g, _):
    n16 = nbuf[pl.ds(g * 16, 16)]
    e16 = ebuff[pl.ds(g * 16, 16)]
    cvals = plsc.load_gather(cnt_v, [e16])
    elig = cvals > 1.5
    cand = jnp.where(elig, e16, -1)
    comb = n16 * 16384 + (cand + 1)
    sk, _sv = plsc.sort_key_val(comb, comb)
    ns = lax.shift_right_logical(sk, 14)
    cs = (sk & 16383) - 1
    _, lastm = plsc.scan_count(ns)
    cur = plsc.load_gather(best_v, [ns])
    plsc.store_scatter(best_v, [ns], jnp.maximum(cur, cs), mask=lastm)
    w16 = lax.shift_right_logical(e16, 7)
    rank, wl = plsc.scan_count(w16)
    curw = plsc.load_gather(cnts_v, [w16])
    plsc.store_scatter(cnts_v, [w16], curw + (rank - bias) + 1, mask=wl)
    return 0

  lax.fori_loop(0, _EPT // 16, _pairs_a, 0)

  carry = 0
  for j in range(4):
    c16 = cnts_v[pl.ds(j * 16, 16)]
    v = (c16 + 15) & -16
    cum = plsc.cumsum(v) + carry
    st = cum - v
    offs_v[pl.ds(j * 16, 16)] = st
    carry = cum[15]
    for i in range(16):
      smem[j * 16 + i] = st[i]
      smem[_NWIN + j * 16 + i] = c16[i]

  def _pairs_b(g, _):
    n16 = nbuf[pl.ds(g * 16, 16)]
    e16 = ebuff[pl.ds(g * 16, 16)]
    w16 = lax.shift_right_logical(e16, 7)
    eloc16 = e16 & (_WIN - 1)
    rank, wl = plsc.scan_count(w16)
    base = plsc.load_gather(offs_v, [w16])
    dest = base + (rank - bias)
    plsc.store_scatter(nlall, [dest], n16)
    plsc.store_scatter(elall, [dest], eloc16)
    plsc.store_scatter(offs_v, [w16], dest + 1, mask=wl)
    return 0

  lax.fori_loop(0, _EPT // 16, _pairs_b, 0)

  pltpu.sync_copy(best_v, stage_sp.at[s])
  plsc.subcore_barrier()
  pltpu.sync_copy(stage_sp.at[:, pl.ds(s * _NPT, _NPT)], mrg_v)

  def _nodes_a(g, _):
    m = mrg_v[0, pl.ds(g * 16, 16)]
    for r in range(1, 16):
      m = jnp.maximum(m, mrg_v[r, pl.ds(g * 16, 16)])
    best_v[pl.ds(g * 16, 16)] = m
    mw = lax.shift_right_logical(jnp.maximum(m, 0), 7)
    valid = m >= 0
    rank, wl = plsc.scan_count(mw, mask=valid)
    curw = plsc.load_gather(ocnts_v, [mw])
    plsc.store_scatter(ocnts_v, [mw], curw + (rank - bias) + 1, mask=wl)
    return 0

  lax.fori_loop(0, _NPT // 16, _nodes_a, 0)

  carry = 0
  for j in range(4):
    c16 = ocnts_v[pl.ds(j * 16, 16)]
    v = (c16 + 15) & -16
    cum = plsc.cumsum(v) + carry
    st = cum - v
    ooffs_v[pl.ds(j * 16, 16)] = st
    carry = cum[15]
    for i in range(16):
      smem[2 * _NWIN + j * 16 + i] = st[i]
      smem[3 * _NWIN + j * 16 + i] = c16[i]

  def _nodes_b(g, onoff):
    m = best_v[pl.ds(g * 16, 16)]
    nodeid = s * _NPT + g * 16 + iot
    cb = plsc.load_gather(cnt_v, [jnp.maximum(m, 0)])
    iv = 1.0 / jnp.maximum(cb, 1.0)
    valid = m >= 0
    mw = lax.shift_right_logical(jnp.maximum(m, 0), 7)
    mloc = jnp.maximum(m, 0) & (_WIN - 1)
    rank, wl = plsc.scan_count(mw, mask=valid)
    base = plsc.load_gather(ooffs_v, [mw])
    dest = base + (rank - bias)
    plsc.store_scatter(onall, [dest], nodeid, mask=valid)
    plsc.store_scatter(oeall, [dest], mloc, mask=valid)
    plsc.store_scatter(ovall, [dest], iv, mask=valid)
    plsc.store_scatter(ooffs_v, [mw], dest + 1, mask=wl)
    mn = m < 0
    pcn = plsc.all_reduce_population_count(mn)[0]
    plsc.store_compressed(onone.at[pl.ds(onoff, 16)], nodeid, mask=mn)
    return onoff + pcn

  cnt_on = lax.fori_loop(0, _NPT // 16, _nodes_b, 0)

  pltpu.sync_copy(cnts_v.at[pl.ds(0, 64)], tabp_sp.at[pl.ds(s * 64, 64)])
  pltpu.sync_copy(ocnts_v.at[pl.ds(0, 64)], tabo_sp.at[pl.ds(s * 64, 64)])
  plsc.subcore_barrier()
  pltpu.sync_copy(tabp_sp, tabpv)
  pltpu.sync_copy(tabo_sp, tabov)

  def _push(w, carry):
    gs, gso, ncp = carry
    wful = jnp.full((16,), w, jnp.int32)
    c16 = plsc.load_gather(tabpv, [iot * 64 + wful])
    c16p = (c16 + 15) & -16
    cum = plsc.cumsum(c16p)
    ex = cum - c16p
    exb[pl.ds(0, 16)] = ex
    myex = plsc.load_gather(exb, [jnp.full((16,), s, jnp.int32)])[0]
    tot = cum[15]
    gtot = (tot + 63) & -64
    smem[256 + w] = gs
    smem[320 + w] = gtot
    lstart = pl.multiple_of(smem[w], 16)
    mycnt = smem[_NWIN + w]
    myoff = pl.multiple_of(gs + myex, 16)

    def _fp(q, n):
      lo = pl.multiple_of(lstart + q * 16, 16)
      go = pl.multiple_of(myoff + q * 16, 16)
      pltpu.async_copy(nlall.at[pl.ds(lo, 16)], gpn_sp.at[pl.ds(go, 16)],
                       sem_l)
      pltpu.async_copy(elall.at[pl.ds(lo, 16)], gpe_sp.at[pl.ds(go, 16)],
                       sem_l)
      return n + 2
    ncp = lax.fori_loop(0, (mycnt + 15) // 16, _fp, ncp)

    o16 = plsc.load_gather(tabov, [iot * 64 + wful])
    o16p = (o16 + 15) & -16
    ocum = plsc.cumsum(o16p)
    oex = ocum - o16p
    exb[pl.ds(0, 16)] = oex
    myoex = plsc.load_gather(exb, [jnp.full((16,), s, jnp.int32)])[0]
    otot = ocum[15]
    smem[384 + w] = gso
    smem[448 + w] = otot
    olstart = pl.multiple_of(smem[2 * _NWIN + w], 16)
    myocnt = smem[3 * _NWIN + w]
    myooff = pl.multiple_of(gso + myoex, 16)

    def _fo(q, n):
      lo = pl.multiple_of(olstart + q * 16, 16)
      go = pl.multiple_of(myooff + q * 16, 16)
      pltpu.async_copy(onall.at[pl.ds(lo, 16)], gon_sp.at[pl.ds(go, 16)],
                       sem_l)
      pltpu.async_copy(oeall.at[pl.ds(lo, 16)], goe_sp.at[pl.ds(go, 16)],
                       sem_l)
      pltpu.async_copy(ovall.at[pl.ds(lo, 16)], gov_sp.at[pl.ds(go, 16)],
                       sem_l)
      return n + 3
    ncp = lax.fori_loop(0, (myocnt + 15) // 16, _fo, ncp)
    return gs + gtot, gso + otot, ncp

  _, _, ncopies = lax.fori_loop(0, _NWIN, _push, (0, 0, 0))

  def _drain(i, _):
    pltpu.make_async_copy(nlall.at[pl.ds(0, 16)],
                          gpn_sp.at[pl.ds(0, 16)], sem_l).wait()
    return 0
  lax.fori_loop(0, ncopies, _drain, 0)
  plsc.subcore_barrier()

  def _none_pass(bi, _):
    bbase = (c * 2 + bi) * _S

    def _none(v, _v):
      nn = onone[pl.ds(v * 16, 16)]
      src = jnp.where(nn >= _S, 0, nn + bbase)
      dst = jnp.where(nn >= _S, _BS, nn + bbase)
      ob16 = ob.at[pl.ds(0, 16)]
      pltpu.async_copy(xt_hbm.at[src], ob16, sem_o).wait()
      pltpu.async_copy(ob16, agg_hbm.at[dst], sem_o).wait()
      return 0
    lax.fori_loop(0, (cnt_on + 15) // 16, _none, 0)
    return 0
  lax.fori_loop(0, 2, _none_pass, 0)

  def _bp(t, _):
    bi = lax.div(t, _NPASS)
    p = t - bi * _NPASS
    w = p * _NT + s
    bbase = (c * 2 + bi) * _S

    def _zero(i, _i):
      for k in range(16):
        acc[i, pl.ds(k * 16, 16)] = zf
      return 0
    lax.fori_loop(0, _WIN + 1, _zero, 0)

    gs = pl.multiple_of(smem[256 + w], 64)
    gc = smem[320 + w]

    def _chunk(q, _q):
      off = pl.multiple_of(gs + q * _PCH, 64)
      pltpu.sync_copy(gpn_sp.at[pl.ds(off, _PCH)], gnb)
      pltpu.sync_copy(gpe_sp.at[pl.ds(off, _PCH)], geb)
      for m in range(_PCH // 16):
        gidx[pl.ds(m * 16, 16)] = gnb[pl.ds(m * 16, 16)] + bbase
      pltpu.async_copy(xt_hbm.at[gidx], rows, sem_g).wait()

      def _addrow(rr, _r):
        el16 = geb[pl.ds(rr * 16, 16)]
        for r in range(16):
          e = el16[r]
          for cg in range(16):
            plsc.addupdate(acc.at[e, pl.ds(cg * 16, 16)],
                           rows[rr * 16 + r, pl.ds(cg * 16, 16)])
        return 0
      lax.fori_loop(0, _PCH // 16, _addrow, 0)
      return 0
    lax.fori_loop(0, (gc + _PCH - 1) // _PCH, _chunk, 0)

    gso = pl.multiple_of(smem[384 + w], 16)
    gco = smem[448 + w]

    def _ochunk(v, _v):
      off = pl.multiple_of(gso + v * 16, 16)
      pltpu.async_copy(gon_sp.at[pl.ds(off, 16)], onb, sem_o)
      pltpu.async_copy(goe_sp.at[pl.ds(off, 16)], oeb, sem_o)
      pltpu.async_copy(gov_sp.at[pl.ds(off, 16)], ovb, sem_o)
      pltpu.make_async_copy(gon_sp.at[pl.ds(off, 16)], onb, sem_o).wait()
      pltpu.make_async_copy(goe_sp.at[pl.ds(off, 16)], oeb, sem_o).wait()
      pltpu.make_async_copy(gov_sp.at[pl.ds(off, 16)], ovb, sem_o).wait()
      nn = onb[...]
      el16 = oeb[...]
      iv16 = ovb[...]
      dst = jnp.where(nn >= _S, _BS, nn + bbase)
      ob16 = ob.at[pl.ds(0, 16)]
      for r in range(16):
        e = el16[r]
        sc = iv16[r]
        for cg in range(16):
          ob[r, pl.ds(cg * 16, 16)] = acc[e, pl.ds(cg * 16, 16)] * sc
      pltpu.async_copy(ob16, agg_hbm.at[dst], sem_o).wait()
      return 0
    lax.fori_loop(0, (gco + 15) // 16, _ochunk, 0)
    return 0

  lax.fori_loop(0, 2 * _NPASS, _bp, 0)


def _sc_gather_scatter(xt_flat, node_idx, edge_idx):
  mesh = plsc.VectorSubcoreMesh(core_axis_name="c", subcore_axis_name="s")
  f = pl.kernel(
      _sc_body,
      out_type=jax.ShapeDtypeStruct((_BS + 64, _F), jnp.float32),
      mesh=mesh,
      compiler_params=pltpu.CompilerParams(needs_layout_passes=False),
      scratch_types=[
          pltpu.VMEM((_EPT,), jnp.int32),
          pltpu.VMEM((16, 128), jnp.int32),
          pltpu.VMEM((_EPT,), jnp.int32),
          pltpu.VMEM((512,), jnp.float32),
          pltpu.VMEM((512,), jnp.float32),
          pltpu.VMEM((_S,), jnp.float32),
          pltpu.VMEM((_S,), jnp.int32),
          pltpu.VMEM((16, _NPT), jnp.int32),
          pltpu.VMEM((_PLSZ,), jnp.int32),
          pltpu.VMEM((_PLSZ,), jnp.int32),
          pltpu.VMEM((_OSZ,), jnp.int32),
          pltpu.VMEM((_OSZ,), jnp.int32),
          pltpu.VMEM((_OSZ,), jnp.float32),
          pltpu.VMEM((_NPT + 16,), jnp.int32),
          pltpu.VMEM((80,), jnp.int32),
          pltpu.VMEM((80,), jnp.int32),
          pltpu.VMEM((80,), jnp.int32),
          pltpu.VMEM((80,), jnp.int32),
          pltpu.VMEM((1024,), jnp.int32),
          pltpu.VMEM((1024,), jnp.int32),
          pltpu.VMEM((32,), jnp.int32),
          pltpu.VMEM((512,), jnp.int32),
          pltpu.VMEM((512,), jnp.int32),
          pltpu.VMEM((512,), jnp.int32),
          pltpu.VMEM((_PCH,), jnp.int32),
          pltpu.VMEM((_PCH,), jnp.int32),
          pltpu.VMEM((_PCH,), jnp.int32),
          pltpu.VMEM((16,), jnp.int32),
          pltpu.VMEM((16,), jnp.int32),
          pltpu.VMEM((16,), jnp.float32),
          pltpu.VMEM((_PCH, _F), jnp.float32),
          pltpu.VMEM((_WIN + 1, _F), jnp.float32),
          pltpu.VMEM((16, _F), jnp.float32),
          pltpu.SMEM((544,), jnp.int32),
          pltpu.VMEM_SHARED((_S,), jnp.float32),
          pltpu.VMEM_SHARED((16, _S), jnp.int32),
          pltpu.VMEM_SHARED((1024,), jnp.int32),
          pltpu.VMEM_SHARED((1024,), jnp.int32),
          pltpu.VMEM_SHARED((_GPSZ,), jnp.int32),
          pltpu.VMEM_SHARED((_GPSZ,), jnp.int32),
          pltpu.VMEM_SHARED((_GOSZ,), jnp.int32),
          pltpu.VMEM_SHARED((_GOSZ,), jnp.int32),
          pltpu.VMEM_SHARED((_GOSZ,), jnp.float32),
          pltpu.SemaphoreType.DMA,
          pltpu.SemaphoreType.DMA,
          pltpu.SemaphoreType.DMA,
      ],
  )
  return f(xt_flat, node_idx, edge_idx)


def kernel(x, hyperedge_index, weight):
  xt = _matmul(x, weight)
  xt_flat = xt.reshape(_BS, _F)
  node_idx = hyperedge_index[0]
  edge_idx = hyperedge_index[1]
  agg_pad = _sc_gather_scatter(xt_flat, node_idx, edge_idx)
  agg = agg_pad[:_BS].reshape(_B, _S, _F)
  loss = _loss(agg_pad[:_BS], xt_flat)
  return agg, loss


# speedup vs baseline: 2.5538x; 1.0029x over previous
import jax
import jax.numpy as jnp
from jax import lax
from jax.experimental import pallas as pl
from jax.experimental.pallas import tpu as pltpu
from jax.experimental.pallas import tpu_sc as plsc


def _mm_body(x_ref, w_ref, o_ref):
  o_ref[0] = jnp.dot(x_ref[0], w_ref[...],
                     preferred_element_type=jnp.float32)


def _matmul(x, w):
  B, S, FIN = x.shape
  F = w.shape[1]
  SB = 512
  return pl.pallas_call(
      _mm_body,
      grid=(B, S // SB),
      in_specs=[
          pl.BlockSpec((1, SB, FIN), lambda b, i: (b, i, 0)),
          pl.BlockSpec((FIN, F), lambda b, i: (0, 0)),
      ],
      out_specs=pl.BlockSpec((1, SB, F), lambda b, i: (b, i, 0)),
      out_shape=jax.ShapeDtypeStruct((B, S, F), jnp.float32),
  )(x, w)


def _loss_body(n_ref, a_ref, x_ref, o_ref):
  i = pl.program_id(0)
  s = jnp.sum(jnp.abs(a_ref[...] - x_ref[...]))
  prev = jnp.where(i == 0, 0.0, o_ref[0, 0])
  tot = prev + s
  o_ref[0, 0] = jnp.where(i == pl.num_programs(0) - 1,
                          tot / n_ref[0], tot)


def _loss(agg_flat, xt_flat):
  N, F = agg_flat.shape
  RB = 1024
  n = jnp.full((1,), float(N * F), dtype=jnp.float32)
  out = pl.pallas_call(
      _loss_body,
      grid=(N // RB,),
      in_specs=[
          pl.BlockSpec(memory_space=pltpu.SMEM),
          pl.BlockSpec((RB, F), lambda i: (i, 0)),
          pl.BlockSpec((RB, F), lambda i: (i, 0)),
      ],
      out_specs=pl.BlockSpec(memory_space=pltpu.SMEM),
      out_shape=jax.ShapeDtypeStruct((1, 1), jnp.float32),
  )(n, agg_flat, xt_flat)
  return out[0, 0]


_B, _S, _F, _E = 4, 8192, 256, 32768
_NT = 16
_EPT = _E // _NT
_NPT = _S // _NT
_WIN = 128
_NWIN = _S // _WIN
_NPASS = _NWIN // _NT
_TRASH_E = _WIN
_BS = _B * _S
_PCH = 64
_PLSZ = _EPT + _NWIN * 16 + 16
_OSZ = _NPT + _NWIN * 16 + 16
_GPSZ = 54272
_GOSZ = 25088
_LANES = 16


def _sc_body(xt_hbm, nidx_hbm, eidx_hbm, agg_hbm,
             nbuf, ebuf2, ebuff, ones_v, zvec,
             cnt_v, best_v, mrg_v,
             nlall, elall, onall, oeall, ovall, onone,
             cnts_v, offs_v, ocnts_v, ooffs_v, tabpv, tabov, exb,
             pnvec, pevec, pobig,
             gnb, geb, gidx, onb, oeb, ovb,
             rows, acc, ob, smem,
             hist_sp, stage_sp, tabp_sp, tabo_sp,
             gpn_sp, gpe_sp, gon_sp, goe_sp, gov_sp,
             sem_g, sem_o, sem_l):
  c = lax.axis_index("c")
  s = lax.axis_index("s")

  pltpu.sync_copy(nidx_hbm.at[pl.ds(s * _EPT, _EPT)], nbuf)
  pltpu.sync_copy(eidx_hbm.at[pl.ds(s * _EPT, _EPT)], ebuff)
  for j in range(16):
    pltpu.sync_copy(eidx_hbm.at[pl.ds(s * _EPT + j * 128, 128)], ebuf2.at[j])

  zf = jnp.zeros((_LANES,), jnp.float32)
  zi = jnp.zeros((_LANES,), jnp.int32)
  padn = zi
  pade = jnp.full((16,), _TRASH_E, jnp.int32)
  padbig = jnp.full((16,), 1 << 20, jnp.int32)

  def _init1(i, _):
    ones_v[pl.ds(i * 16, 16)] = jnp.ones((16,), jnp.float32)
    zvec[pl.ds(i * 16, 16)] = zf
    pnvec[pl.ds(i * 16, 16)] = padn
    pevec[pl.ds(i * 16, 16)] = pade
    pobig[pl.ds(i * 16, 16)] = padbig
    return 0
  lax.fori_loop(0, 32, _init1, 0)

  def _init2(i, _):
    best_v[pl.ds(i * 16, 16)] = jnp.full((16,), -1, jnp.int32)
    return 0
  lax.fori_loop(0, 512, _init2, 0)

  def _init3(i, _):
    cnts_v[pl.ds(i * 16, 16)] = zi
    offs_v[pl.ds(i * 16, 16)] = zi
    ocnts_v[pl.ds(i * 16, 16)] = zi
    ooffs_v[pl.ds(i * 16, 16)] = zi
    return 0
  lax.fori_loop(0, 5, _init3, 0)

  def _init4(i, _):
    nlall[pl.ds(i * 16, 16)] = padn
    elall[pl.ds(i * 16, 16)] = pade
    return 0
  lax.fori_loop(0, _PLSZ // 16, _init4, 0)

  def _init5(i, _):
    onall[pl.ds(i * 16, 16)] = padbig
    oeall[pl.ds(i * 16, 16)] = pade
    ovall[pl.ds(i * 16, 16)] = zf
    return 0
  lax.fori_loop(0, _OSZ // 16, _init5, 0)

  def _init6(i, _):
    onone[pl.ds(i * 16, 16)] = padbig
    return 0
  lax.fori_loop(0, (_NPT + 16) // 16, _init6, 0)

  gp_stripe = _GPSZ // 16
  go_stripe = _GOSZ // 16
  for j in range(7):
    sz = min(512, gp_stripe - j * 512)
    if sz > 0:
      pltpu.sync_copy(pnvec.at[pl.ds(0, sz)],
                      gpn_sp.at[pl.ds(s * gp_stripe + j * 512, sz)])
      pltpu.sync_copy(pevec.at[pl.ds(0, sz)],
                      gpe_sp.at[pl.ds(s * gp_stripe + j * 512, sz)])
  for j in range(4):
    osz = min(512, go_stripe - j * 512)
    if osz > 0:
      pltpu.sync_copy(pobig.at[pl.ds(0, osz)],
                      gon_sp.at[pl.ds(s * go_stripe + j * 512, osz)])
      pltpu.sync_copy(pevec.at[pl.ds(0, osz)],
                      goe_sp.at[pl.ds(s * go_stripe + j * 512, osz)])
      pltpu.sync_copy(zvec.at[pl.ds(0, osz)],
                      gov_sp.at[pl.ds(s * go_stripe + j * 512, osz)])

  pltpu.sync_copy(zvec, hist_sp.at[pl.ds(s * 512, 512)])
  plsc.subcore_barrier()

  def _hist(j, _):
    pltpu.sync_copy(ones_v.at[pl.ds(0, 128)], hist_sp.at[ebuf2.at[j]],
                    add=True)
    return 0
  lax.fori_loop(0, 16, _hist, 0)
  plsc.subcore_barrier()

  pltpu.sync_copy(hist_sp, cnt_v)

  iot = lax.iota(jnp.int32, _LANES)
  rz, _ = plsc.scan_count(zi)
  bias = rz[15] - 15

  def _pairs_a(g, _):
    n16 = nbuf[pl.ds(g * 16, 16)]
    e16 = ebuff[pl.ds(g * 16, 16)]
    cvals = plsc.load_gather(cnt_v, [e16])
    elig = cvals > 1.5
    cand = jnp.where(elig, e16, -1)
    comb = n16 * 16384 + (cand + 1)
    sk, _sv = plsc.sort_key_val(comb, comb)
    ns = lax.shift_right_logical(sk, 14)
    cs = (sk & 16383) - 1
    _, lastm = plsc.scan_count(ns)
    cur = plsc.load_gather(best_v, [ns])
    plsc.store_scatter(best_v, [ns], jnp.maximum(cur, cs), mask=lastm)
    w16 = lax.shift_right_logical(e16, 7)
    rank, wl = plsc.scan_count(w16)
    curw = plsc.load_gather(cnts_v, [w16])
    plsc.store_scatter(cnts_v, [w16], curw + (rank - bias) + 1, mask=wl)
    return 0

  lax.fori_loop(0, _EPT // 16, _pairs_a, 0)

  carry = 0
  for j in range(4):
    c16 = cnts_v[pl.ds(j * 16, 16)]
    v = (c16 + 15) & -16
    cum = plsc.cumsum(v) + carry
    st = cum - v
    offs_v[pl.ds(j * 16, 16)] = st
    carry = cum[15]
    for i in range(16):
      smem[j * 16 + i] = st[i]
      smem[_NWIN + j * 16 + i] = c16[i]

  def _pairs_b(g, _):
    n16 = nbuf[pl.ds(g * 16, 16)]
    e16 = ebuff[pl.ds(g * 16, 16)]
    w16 = lax.shift_right_logical(e16, 7)
    eloc16 = e16 & (_WIN - 1)
    rank, wl = plsc.scan_count(w16)
    base = plsc.load_gather(offs_v, [w16])
    dest = base + (rank - bias)
    plsc.store_scatter(nlall, [dest], n16)
    plsc.store_scatter(elall, [dest], eloc16)
    plsc.store_scatter(offs_v, [w16], dest + 1, mask=wl)
    return 0

  lax.fori_loop(0, _EPT // 16, _pairs_b, 0)

  pltpu.sync_copy(best_v, stage_sp.at[s])
  plsc.subcore_barrier()
  pltpu.sync_copy(stage_sp.at[:, pl.ds(s * _NPT, _NPT)], mrg_v)

  def _nodes_a(g, _):
    m = mrg_v[0, pl.ds(g * 16, 16)]
    for r in range(1, 16):
      m = jnp.maximum(m, mrg_v[r, pl.ds(g * 16, 16)])
    best_v[pl.ds(g * 16, 16)] = m
    mw = lax.shift_right_logical(jnp.maximum(m, 0), 7)
    valid = m >= 0
    rank, wl = plsc.scan_count(mw, mask=valid)
    curw = plsc.load_gather(ocnts_v, [mw])
    plsc.store_scatter(ocnts_v, [mw], curw + (rank - bias) + 1, mask=wl)
    return 0

  lax.fori_loop(0, _NPT // 16, _nodes_a, 0)

  carry = 0
  for j in range(4):
    c16 = ocnts_v[pl.ds(j * 16, 16)]
    v = (c16 + 15) & -16
    cum = plsc.cumsum(v) + carry
    st = cum - v
    ooffs_v[pl.ds(j * 16, 16)] = st
    carry = cum[15]
    for i in range(16):
      smem[2 * _NWIN + j * 16 + i] = st[i]
      smem[3 * _NWIN + j * 16 + i] = c16[i]

  def _nodes_b(g, onoff):
    m = best_v[pl.ds(g * 16, 16)]
    nodeid = s * _NPT + g * 16 + iot
    cb = plsc.load_gather(cnt_v, [jnp.maximum(m, 0)])
    iv = 1.0 / jnp.maximum(cb, 1.0)
    valid = m >= 0
    mw = lax.shift_right_logical(jnp.maximum(m, 0), 7)
    mloc = jnp.maximum(m, 0) & (_WIN - 1)
    rank, wl = plsc.scan_count(mw, mask=valid)
    base = plsc.load_gather(ooffs_v, [mw])
    dest = base + (rank - bias)
    plsc.store_scatter(onall, [dest], nodeid, mask=valid)
    plsc.store_scatter(oeall, [dest], mloc, mask=valid)
    plsc.store_scatter(ovall, [dest], iv, mask=valid)
    plsc.store_scatter(ooffs_v, [mw], dest + 1, mask=wl)
    mn = m < 0
    pcn = plsc.all_reduce_population_count(mn)[0]
    plsc.store_compressed(onone.at[pl.ds(onoff, 16)], nodeid, mask=mn)
    return onoff + pcn

  cnt_on = lax.fori_loop(0, _NPT // 16, _nodes_b, 0)

  pltpu.sync_copy(cnts_v.at[pl.ds(0, 64)], tabp_sp.at[pl.ds(s * 64, 64)])
  pltpu.sync_copy(ocnts_v.at[pl.ds(0, 64)], tabo_sp.at[pl.ds(s * 64, 64)])
  plsc.subcore_barrier()
  pltpu.sync_copy(tabp_sp, tabpv)
  pltpu.sync_copy(tabo_sp, tabov)

  def _push(w, carry):
    gs, gso, ncp = carry
    wful = jnp.full((16,), w, jnp.int32)
    c16 = plsc.load_gather(tabpv, [iot * 64 + wful])
    c16p = (c16 + 15) & -16
    cum = plsc.cumsum(c16p)
    ex = cum - c16p
    exb[pl.ds(0, 16)] = ex
    myex = plsc.load_gather(exb, [jnp.full((16,), s, jnp.int32)])[0]
    tot = cum[15]
    gtot = (tot + 63) & -64
    smem[256 + w] = gs
    smem[320 + w] = gtot
    lstart = pl.multiple_of(smem[w], 16)
    mycnt = smem[_NWIN + w]
    myoff = pl.multiple_of(gs + myex, 16)

    def _fp(q, n):
      lo = pl.multiple_of(lstart + q * 16, 16)
      go = pl.multiple_of(myoff + q * 16, 16)
      pltpu.async_copy(nlall.at[pl.ds(lo, 16)], gpn_sp.at[pl.ds(go, 16)],
                       sem_l)
      pltpu.async_copy(elall.at[pl.ds(lo, 16)], gpe_sp.at[pl.ds(go, 16)],
                       sem_l)
      return n + 2
    ncp = lax.fori_loop(0, (mycnt + 15) // 16, _fp, ncp)

    o16 = plsc.load_gather(tabov, [iot * 64 + wful])
    o16p = (o16 + 15) & -16
    ocum = plsc.cumsum(o16p)
    oex = ocum - o16p
    exb[pl.ds(0, 16)] = oex
    myoex = plsc.load_gather(exb, [jnp.full((16,), s, jnp.int32)])[0]
    otot = ocum[15]
    smem[384 + w] = gso
    smem[448 + w] = otot
    olstart = pl.multiple_of(smem[2 * _NWIN + w], 16)
    myocnt = smem[3 * _NWIN + w]
    myooff = pl.multiple_of(gso + myoex, 16)

    def _fo(q, n):
      lo = pl.multiple_of(olstart + q * 16, 16)
      go = pl.multiple_of(myooff + q * 16, 16)
      pltpu.async_copy(onall.at[pl.ds(lo, 16)], gon_sp.at[pl.ds(go, 16)],
                       sem_l)
      pltpu.async_copy(oeall.at[pl.ds(lo, 16)], goe_sp.at[pl.ds(go, 16)],
                       sem_l)
      pltpu.async_copy(ovall.at[pl.ds(lo, 16)], gov_sp.at[pl.ds(go, 16)],
                       sem_l)
      return n + 3
    ncp = lax.fori_loop(0, (myocnt + 15) // 16, _fo, ncp)
    return gs + gtot, gso + otot, ncp

  _, _, ncopies = lax.fori_loop(0, _NWIN, _push, (0, 0, 0))

  def _drain(i, _):
    pltpu.make_async_copy(nlall.at[pl.ds(0, 16)],
                          gpn_sp.at[pl.ds(0, 16)], sem_l).wait()
    return 0
  lax.fori_loop(0, ncopies, _drain, 0)
  plsc.subcore_barrier()

  def _none_pass(bi, _):
    bbase = (c * 2 + bi) * _S

    def _none(v, _v):
      nn = onone[pl.ds(v * 16, 16)]
      src = jnp.where(nn >= _S, 0, nn + bbase)
      dst = jnp.where(nn >= _S, _BS, nn + bbase)
      ob16 = ob.at[pl.ds(0, 16)]
      pltpu.async_copy(xt_hbm.at[src], ob16, sem_o).wait()
      pltpu.async_copy(ob16, agg_hbm.at[dst], sem_o).wait()
      return 0
    lax.fori_loop(0, (cnt_on + 15) // 16, _none, 0)
    return 0
  lax.fori_loop(0, 2, _none_pass, 0)

  def _bp(t, _):
    bi = lax.div(t, _NPASS)
    p = t - bi * _NPASS
    w = p * _NT + s
    bbase = (c * 2 + bi) * _S

    def _zero(i, _i):
      for k in range(16):
        acc[i, pl.ds(k * 16, 16)] = zf
      return 0
    lax.fori_loop(0, _WIN + 1, _zero, 0)

    gs = pl.multiple_of(smem[256 + w], 64)
    gc = smem[320 + w]

    def _slab(sl, _s2):
      so = pl.multiple_of(gs + sl * 2048, 64)
      d1 = pltpu.async_copy(gpn_sp.at[pl.ds(so, 2048)], gnb, sem_l)
      d2 = pltpu.async_copy(gpe_sp.at[pl.ds(so, 2048)], geb, sem_l)
      d1.wait()
      d2.wait()
      nch = (jnp.minimum(gc - sl * 2048, 2048) + _PCH - 1) // _PCH

      def _chunk(q, _q):
        base = q * _PCH
        for m in range(_PCH // 16):
          gidx[pl.ds(m * 16, 16)] = gnb[pl.ds(base + m * 16, 16)] + bbase
        pltpu.async_copy(xt_hbm.at[gidx], rows, sem_g).wait()

        def _addrow(rr, _r):
          el16 = geb[pl.ds(base + rr * 16, 16)]
          for r in range(16):
            e = el16[r]
            for cg in range(16):
              plsc.addupdate(acc.at[e, pl.ds(cg * 16, 16)],
                             rows[rr * 16 + r, pl.ds(cg * 16, 16)])
          return 0
        lax.fori_loop(0, _PCH // 16, _addrow, 0)
        return 0
      lax.fori_loop(0, nch, _chunk, 0)
      return 0
    lax.fori_loop(0, (gc + 2047) // 2048, _slab, 0)

    gso = pl.multiple_of(smem[384 + w], 16)
    gco = smem[448 + w]

    def _oslab(sl, _s2):
      so = pl.multiple_of(gso + sl * 512, 16)
      d1 = pltpu.async_copy(gon_sp.at[pl.ds(so, 512)], onb, sem_l)
      d2 = pltpu.async_copy(goe_sp.at[pl.ds(so, 512)], oeb, sem_l)
      d3 = pltpu.async_copy(gov_sp.at[pl.ds(so, 512)], ovb, sem_l)
      d1.wait()
      d2.wait()
      d3.wait()
      nch = (jnp.minimum(gco - sl * 512, 512) + 15) // 16

      def _ochunk(v, _v):
        off = v * 16
        nn = onb[pl.ds(off, 16)]
        el16 = oeb[pl.ds(off, 16)]
        iv16 = ovb[pl.ds(off, 16)]
        dst = jnp.where(nn >= _S, _BS, nn + bbase)
        ob16 = ob.at[pl.ds(0, 16)]
        for r in range(16):
          e = el16[r]
          sc = iv16[r]
          for cg in range(16):
            ob[r, pl.ds(cg * 16, 16)] = acc[e, pl.ds(cg * 16, 16)] * sc
        pltpu.async_copy(ob16, agg_hbm.at[dst], sem_o).wait()
        return 0
      lax.fori_loop(0, nch, _ochunk, 0)
      return 0
    lax.fori_loop(0, (gco + 511) // 512, _oslab, 0)
    return 0

  lax.fori_loop(0, 2 * _NPASS, _bp, 0)


def _sc_gather_scatter(xt_flat, node_idx, edge_idx):
  mesh = plsc.VectorSubcoreMesh(core_axis_name="c", subcore_axis_name="s")
  f = pl.kernel(
      _sc_body,
      out_type=jax.ShapeDtypeStruct((_BS + 64, _F), jnp.float32),
      mesh=mesh,
      compiler_params=pltpu.CompilerParams(needs_layout_passes=False),
      scratch_types=[
          pltpu.VMEM((_EPT,), jnp.int32),
          pltpu.VMEM((16, 128), jnp.int32),
          pltpu.VMEM((_EPT,), jnp.int32),
          pltpu.VMEM((512,), jnp.float32),
          pltpu.VMEM((512,), jnp.float32),
          pltpu.VMEM((_S,), jnp.float32),
          pltpu.VMEM((_S,), jnp.int32),
          pltpu.VMEM((16, _NPT), jnp.int32),
          pltpu.VMEM((_PLSZ,), jnp.int32),
          pltpu.VMEM((_PLSZ,), jnp.int32),
          pltpu.VMEM((_OSZ,), jnp.int32),
          pltpu.VMEM((_OSZ,), jnp.int32),
          pltpu.VMEM((_OSZ,), jnp.float32),
          pltpu.VMEM((_NPT + 16,), jnp.int32),
          pltpu.VMEM((80,), jnp.int32),
          pltpu.VMEM((80,), jnp.int32),
          pltpu.VMEM((80,), jnp.int32),
          pltpu.VMEM((80,), jnp.int32),
          pltpu.VMEM((1024,), jnp.int32),
          pltpu.VMEM((1024,), jnp.int32),
          pltpu.VMEM((32,), jnp.int32),
          pltpu.VMEM((512,), jnp.int32),
          pltpu.VMEM((512,), jnp.int32),
          pltpu.VMEM((512,), jnp.int32),
          pltpu.VMEM((2048,), jnp.int32),
          pltpu.VMEM((2048,), jnp.int32),
          pltpu.VMEM((_PCH,), jnp.int32),
          pltpu.VMEM((512,), jnp.int32),
          pltpu.VMEM((512,), jnp.int32),
          pltpu.VMEM((512,), jnp.float32),
          pltpu.VMEM((_PCH, _F), jnp.float32),
          pltpu.VMEM((_WIN + 1, _F), jnp.float32),
          pltpu.VMEM((16, _F), jnp.float32),
          pltpu.SMEM((544,), jnp.int32),
          pltpu.VMEM_SHARED((_S,), jnp.float32),
          pltpu.VMEM_SHARED((16, _S), jnp.int32),
          pltpu.VMEM_SHARED((1024,), jnp.int32),
          pltpu.VMEM_SHARED((1024,), jnp.int32),
          pltpu.VMEM_SHARED((_GPSZ,), jnp.int32),
          pltpu.VMEM_SHARED((_GPSZ,), jnp.int32),
          pltpu.VMEM_SHARED((_GOSZ,), jnp.int32),
          pltpu.VMEM_SHARED((_GOSZ,), jnp.int32),
          pltpu.VMEM_SHARED((_GOSZ,), jnp.float32),
          pltpu.SemaphoreType.DMA,
          pltpu.SemaphoreType.DMA,
          pltpu.SemaphoreType.DMA,
      ],
  )
  return f(xt_flat, node_idx, edge_idx)


def kernel(x, hyperedge_index, weight):
  xt = _matmul(x, weight)
  xt_flat = xt.reshape(_BS, _F)
  node_idx = hyperedge_index[0]
  edge_idx = hyperedge_index[1]
  agg_pad = _sc_gather_scatter(xt_flat, node_idx, edge_idx)
  agg = agg_pad[:_BS].reshape(_B, _S, _F)
  loss = _loss(agg_pad[:_BS], xt_flat)
  return agg, loss
